# Initial kernel scaffold; baseline (speedup 1.0000x reference)
import jax
import jax.numpy as jnp
from jax.experimental import pallas as pl


def kernel(positions, species, pairs, W1, b1, W2, alch):
    raise NotImplementedError("write your pallas kernel here")


# speedup vs baseline: 15.3180x; 15.3180x over previous
import functools
import numpy as np

import jax
import jax.numpy as jnp
from jax.experimental import pallas as pl
from jax.experimental.pallas import tpu as pltpu

N_ATOMS = 10000
NSP = 4
Q = 4
NMAX = 6
LMAX = 3
RC = 5.0
H = 32

EBLK = 3200


def _edge_kernel(psrc_ref, pdst_ref, wq_ref, w1_ref, b1_ref, w2_ref, g_ref):
    rx = pdst_ref[0, :] - psrc_ref[0, :]
    ry = pdst_ref[1, :] - psrc_ref[1, :]
    rz = pdst_ref[2, :] - psrc_ref[2, :]
    d2 = rx * rx + ry * ry + rz * rz + 1e-12
    dist = jnp.sqrt(d2)
    inv = 1.0 / dist
    x = rx * inv
    y = ry * inv
    z = rz * inv
    fc = 0.5 * (jnp.cos(jnp.pi * jnp.minimum(dist, RC) / RC) + 1.0)
    fc = jnp.where(dist < RC, fc, 0.0)

    h = jnp.tanh(dist[None, :] * w1_ref[0, :][:, None] + b1_ref[0, :][:, None])
    rn = jax.lax.dot_general(
        w2_ref[...], h, (((0,), (0,)), ((), ())),
        preferred_element_type=jnp.float32)
    rn = rn * fc[None, :]

    c0 = 0.28209479177387814
    ys = [
        jnp.full_like(x, c0),
        0.4886025119029199 * y,
        0.4886025119029199 * z,
        0.4886025119029199 * x,
        1.0925484305920792 * x * y,
        1.0925484305920792 * y * z,
        0.31539156525252005 * (3.0 * z * z - 1.0),
        1.0925484305920792 * x * z,
        0.5462742152960396 * (x * x - y * y),
        0.5900435899266435 * y * (3.0 * x * x - y * y),
        2.890611442640554 * x * y * z,
        0.4570457994644658 * y * (5.0 * z * z - 1.0),
        0.3731763325901154 * z * (5.0 * z * z - 3.0),
        0.4570457994644658 * x * (5.0 * z * z - 1.0),
        1.445305721320277 * z * (x * x - y * y),
        0.5900435899266435 * x * (x * x - 3.0 * y * y),
    ]
    yall = jnp.stack(ys, axis=0)

    moff = [0, 1, 4, 9]
    rows = []
    for l in range(LMAX + 1):
        m = 2 * l + 1
        rl = rn[l * NMAX:(l + 1) * NMAX, :]
        yl = yall[moff[l]:moff[l] + m, :]
        rows.append((rl[:, None, :] * yl[None, :, :]).reshape(NMAX * m, EBLK))
    ry = jnp.concatenate(rows, axis=0)
    wq = wq_ref[...]
    g = wq[:, None, :] * ry[None, :, :]
    g_ref[...] = g.reshape(Q * 96, EBLK)


def _edge_features(psrc_t, pdst_t, wq_t, W1, b1, W2, n_edges):
    nblk = n_edges // EBLK
    grid = (nblk,)
    return pl.pallas_call(
        _edge_kernel,
        grid=grid,
        in_specs=[
            pl.BlockSpec((3, EBLK), lambda i: (0, i)),
            pl.BlockSpec((3, EBLK), lambda i: (0, i)),
            pl.BlockSpec((Q, EBLK), lambda i: (0, i)),
            pl.BlockSpec((1, H), lambda i: (0, 0)),
            pl.BlockSpec((1, H), lambda i: (0, 0)),
            pl.BlockSpec((H, (LMAX + 1) * NMAX), lambda i: (0, 0)),
        ],
        out_specs=pl.BlockSpec((Q * 96, EBLK), lambda i: (0, i)),
        out_shape=jax.ShapeDtypeStruct((Q * 96, n_edges), jnp.float32),
    )(psrc_t, pdst_t, wq_t, W1, b1, W2)


def kernel(positions, species, pairs, W1, b1, W2, alch):
    src = pairs[0]
    dst = pairs[1]
    E = src.shape[0]
    psrc_t = positions[src].T
    pdst_t = positions[dst].T
    wq_t = alch[species[dst]].T

    g_t = _edge_features(psrc_t, pdst_t, wq_t, W1, b1.reshape(1, H), W2, E)
    coeff = jax.ops.segment_sum(g_t.T, src, num_segments=N_ATOMS)
    coeff = coeff.reshape(N_ATOMS, Q, 96)

    moff = [0, 1, 4, 9]
    feats = []
    for l in range(LMAX + 1):
        m = 2 * l + 1
        cl = coeff[:, :, NMAX * moff[l]:NMAX * (moff[l] + m)].reshape(
            N_ATOMS, Q, NMAX, m)
        ps = jnp.einsum('iqnm,ipkm->iqnpk', cl, cl) / np.sqrt(2 * l + 1)
        feats.append(ps.reshape(N_ATOMS, -1))
    return jnp.concatenate(feats, axis=-1)


# speedup vs baseline: 33.0804x; 2.1596x over previous
import numpy as np

import jax
import jax.numpy as jnp
from jax import lax
from jax.experimental import pallas as pl
from jax.experimental.pallas import tpu as pltpu
from jax.experimental.pallas import tpu_sc as plsc

N_ATOMS = 10000
E_EDGES = 160000
Q = 4
NMAX = 6
LMAX = 3
RC = 5.0
H = 32
MOFF = (0, 1, 4, 9)
F96 = 96
FTOT = Q * F96
NPS = 24
OUT_W = (LMAX + 1) * NPS * NPS

NC, NS = 2, 16
NW = NC * NS
ROWS = E_EDGES // 128
ROWS_A = ROWS // NW
EXTRA_A = ROWS - ROWS_A * NW
ROWS_C = ROWS // NS
EXTRA_C = ROWS - ROWS_C * NS
ZCH = 624
HROWS = ROWS // NC

_f32 = jnp.float32
_i32 = jnp.int32


def _np_consts():
    c0 = 0.28209479177387814
    c1 = 0.4886025119029199
    c2a, c2b, c2c = 1.0925484305920792, 0.31539156525252005, 0.5462742152960396
    c3a, c3b, c3c, c3d = (0.5900435899266435, 2.890611442640554,
                          0.4570457994644658, 0.3731763325901154)
    yc = np.zeros((20, 16), np.float32)
    yc[0, 0] = c0
    yc[2, 1] = c1
    yc[3, 2] = c1
    yc[1, 3] = c1
    yc[4, 4] = c2a
    yc[5, 5] = c2a
    yc[9, 6] = 3.0 * c2b
    yc[0, 6] = -c2b
    yc[6, 7] = c2a
    yc[7, 8] = c2c
    yc[8, 8] = -c2c
    yc[10, 9] = 3.0 * c3a
    yc[11, 9] = -c3a
    yc[12, 10] = c3b
    yc[13, 11] = 5.0 * c3c
    yc[2, 11] = -c3c
    yc[14, 12] = 5.0 * c3d
    yc[3, 12] = -3.0 * c3d
    yc[15, 13] = 5.0 * c3c
    yc[1, 13] = -c3c
    yc[16, 14] = 1.445305721320277
    yc[17, 14] = -1.445305721320277
    yc[18, 15] = c3a
    yc[19, 15] = -3.0 * c3a

    sa = np.zeros((Q, F96), np.float32)
    sb = np.zeros((NPS, F96), np.float32)
    for q in range(Q):
        for ln in range(NPS):
            sa[q, q * NPS + ln] = 1.0
            sb[ln, q * NPS + ln] = 1.0
    snm = np.zeros((F96, FTOT), np.float32)
    sm = np.zeros((16, FTOT), np.float32)
    for q in range(Q):
        for l in range(LMAX + 1):
            m = 2 * l + 1
            for n in range(NMAX):
                for mm in range(m):
                    col = q * F96 + 6 * MOFF[l] + n * m + mm
                    snm[q * NPS + l * NMAX + n, col] = 1.0
                    sm[MOFF[l] + mm, col] = 1.0
    pmat = np.zeros((FTOT, FTOT), np.float32)
    plane = 0
    for l in range(LMAX + 1):
        m = 2 * l + 1
        for mm in range(m):
            for q in range(Q):
                for n in range(NMAX):
                    col = q * F96 + 6 * MOFF[l] + n * m + mm
                    pmat[col, plane * NPS + q * NMAX + n] = 1.0
            plane += 1
    rsel = np.zeros((NPS, NPS * NPS), np.float32)
    tsel = np.zeros((NPS, NPS * NPS), np.float32)
    for a in range(NPS):
        for b in range(NPS):
            rsel[a, a * NPS + b] = 1.0
            tsel[b, a * NPS + b] = 1.0
    return yc, sa, sb, snm, sm, pmat, rsel, tsel


_YC, _SA, _SB, _SNM, _SM, _PMAT, _RSEL, _TSEL = _np_consts()


def _mm(a, b):
    return lax.dot_general(a, b, (((1,), (0,)), ((), ())),
                           preferred_element_type=_f32,
                           precision=lax.Precision.HIGHEST)


def _gather_body(recf, src3d, dst3d, rsrc, rdst,
                 table, sidx, didx, srow, drow):
    wid = lax.axis_index("s") * NC + lax.axis_index("c")
    pltpu.sync_copy(recf, table)
    lanes = lax.iota(_i32, 16)

    def do_row(row):
        pltpu.sync_copy(src3d.at[row], sidx)
        pltpu.sync_copy(dst3d.at[row], didx)
        for k in range(8):
            s16 = sidx[0, pl.ds(k * 16, 16)] * 8
            d16 = didx[0, pl.ds(k * 16, 16)] * 8
            sl4 = (lanes + k * 16) * 4
            dl8 = (lanes + k * 16) * 8
            for comp in range(3):
                vs = plsc.load_gather(table, [s16 + comp])
                plsc.store_scatter(srow, [sl4 + comp], vs)
            for comp in (0, 1, 2, 4, 5, 6, 7):
                vd = plsc.load_gather(table, [d16 + comp])
                plsc.store_scatter(drow, [dl8 + comp], vd)
        pltpu.sync_copy(srow, rsrc.at[pl.ds(row * 512, 512)])
        pltpu.sync_copy(drow, rdst.at[pl.ds(row * 1024, 1024)])

    def body(i, _):
        do_row(wid * ROWS_A + i)
        return 0

    lax.fori_loop(0, ROWS_A, body, 0)

    @pl.when(wid < EXTRA_A)
    def _():
        do_row(NW * ROWS_A + wid)


def _sc_gather(recf, src3d, dst3d):
    mesh = plsc.VectorSubcoreMesh(core_axis_name="c", subcore_axis_name="s")
    f = pl.kernel(
        _gather_body,
        out_type=[jax.ShapeDtypeStruct((E_EDGES * 4,), _f32),
                  jax.ShapeDtypeStruct((E_EDGES * 8,), _f32)],
        mesh=mesh,
        scratch_types=[pltpu.VMEM((N_ATOMS * 8,), _f32),
                       pltpu.VMEM((1, 128), _i32),
                       pltpu.VMEM((1, 128), _i32),
                       pltpu.VMEM((512,), _f32),
                       pltpu.VMEM((1024,), _f32)],
        compiler_params=pltpu.CompilerParams(needs_layout_passes=False),
    )
    return f(recf, src3d, dst3d)


EBLK = 640
IBLK = EBLK // 128


def _edge_body(rsrc_ref, rdst_ref, w1_ref, b1_ref, w2_ref,
               yc_ref, sa_ref, sb_ref, snm_ref, sm_ref, g_ref):
    rs = rsrc_ref[...]
    rd = rdst_ref[...]
    rx = rd[:, 0:1] - rs[:, 0:1]
    ry = rd[:, 1:2] - rs[:, 1:2]
    rz = rd[:, 2:3] - rs[:, 2:3]
    d2 = rx * rx + ry * ry + rz * rz + 1e-12
    dist = jnp.sqrt(d2)
    inv = 1.0 / dist
    fc = 0.5 * (jnp.cos(jnp.pi * jnp.minimum(dist, RC) / RC) + 1.0)
    fc = jnp.where(dist < RC, fc, 0.0)

    h = jnp.tanh(dist * w1_ref[...] + b1_ref[...])
    rn = _mm(h, w2_ref[...]) * fc

    x = rx * inv
    y = ry * inv
    z = rz * inv
    xx, yy, zz = x * x, y * y, z * z
    xy, yz, xz = x * y, y * z, x * z
    mono = (jnp.ones_like(x), x, y, z, xy, yz, xz, xx, yy, zz,
            xx * y, yy * y, xy * z, yz * z, zz * z, x * zz,
            xx * z, yy * z, xx * x, x * yy)
    yc = yc_ref[...]
    ysph = mono[0] * yc[0:1, :]
    for t in range(1, 20):
        ysph = ysph + mono[t] * yc[t:t + 1, :]

    wq = rd[:, 4:8]
    rnq = _mm(wq, sa_ref[...]) * _mm(rn, sb_ref[...])
    g = _mm(rnq, snm_ref[...]) * _mm(ysph, sm_ref[...])
    g_ref[...] = g


def _tc_edges(rsrc, rdst, W1, b1, W2):
    nblk = E_EDGES // EBLK
    consts = [jnp.asarray(a) for a in (_YC, _SA, _SB, _SNM, _SM)]
    cspecs = [pl.BlockSpec(a.shape, lambda i: (0, 0)) for a in consts]
    return pl.pallas_call(
        _edge_body,
        grid=(nblk,),
        in_specs=[
            pl.BlockSpec((EBLK, 4), lambda i: (i, 0)),
            pl.BlockSpec((EBLK, 8), lambda i: (i, 0)),
            pl.BlockSpec((1, H), lambda i: (0, 0)),
            pl.BlockSpec((1, H), lambda i: (0, 0)),
            pl.BlockSpec((H, NPS), lambda i: (0, 0)),
        ] + cspecs,
        out_specs=pl.BlockSpec((EBLK, FTOT), lambda i: (i, 0)),
        out_shape=jax.ShapeDtypeStruct((E_EDGES, FTOT), _f32),
    )(rsrc, rdst, W1, b1, W2, *consts)


def _scatter_body(g_hbm, src3d, zeros_hbm, coeff_hbm, part_hbm,
                  idxrow, gbuf, acc):
    c = lax.axis_index("c")
    t = lax.axis_index("s")

    def zero_acc():
        pltpu.sync_copy(zeros_hbm, acc.at[pl.ds(t * ZCH, ZCH)])

        @pl.when(t == 0)
        def _():
            pltpu.sync_copy(zeros_hbm.at[pl.ds(0, N_ATOMS - NS * ZCH)],
                            acc.at[pl.ds(NS * ZCH, N_ATOMS - NS * ZCH)])

    def do_row(row, col):
        pltpu.sync_copy(src3d.at[row], idxrow)
        pltpu.sync_copy(g_hbm.at[pl.ds(row * 128, 128), pl.ds(col, 128)],
                        gbuf)
        pltpu.sync_copy(gbuf, acc.at[idxrow.at[0]], add=True)

    zero_acc()
    plsc.subcore_barrier()

    def body1(i, _):
        do_row(t * ROWS_C + i, c * 128)
        return 0

    lax.fori_loop(0, ROWS_C, body1, 0)

    @pl.when(t < EXTRA_C)
    def _():
        do_row(NS * ROWS_C + t, c * 128)

    plsc.subcore_barrier()
    pltpu.sync_copy(acc.at[pl.ds(t * ZCH, ZCH)],
                    coeff_hbm.at[pl.ds(t * ZCH, ZCH), pl.ds(c * 128, 128)])

    @pl.when(t == 0)
    def _():
        pltpu.sync_copy(
            acc.at[pl.ds(NS * ZCH, N_ATOMS - NS * ZCH)],
            coeff_hbm.at[pl.ds(NS * ZCH, N_ATOMS - NS * ZCH),
                         pl.ds(c * 128, 128)])

    plsc.subcore_barrier()
    zero_acc()
    plsc.subcore_barrier()
    hrows = HROWS // NS

    def body2(i, _):
        do_row(c * HROWS + t * hrows + i, 2 * 128)
        return 0

    lax.fori_loop(0, hrows, body2, 0)

    @pl.when(t < HROWS - NS * hrows)
    def _():
        do_row(c * HROWS + NS * hrows + t, 2 * 128)

    plsc.subcore_barrier()
    pltpu.sync_copy(acc.at[pl.ds(t * ZCH, ZCH)],
                    part_hbm.at[c, pl.ds(t * ZCH, ZCH)])

    @pl.when(t == 0)
    def _():
        pltpu.sync_copy(acc.at[pl.ds(NS * ZCH, N_ATOMS - NS * ZCH)],
                        part_hbm.at[c, pl.ds(NS * ZCH, N_ATOMS - NS * ZCH)])


def _sc_scatter(g, src3d, zeros):
    mesh = plsc.VectorSubcoreMesh(core_axis_name="c", subcore_axis_name="s")
    f = pl.kernel(
        _scatter_body,
        out_type=[jax.ShapeDtypeStruct((N_ATOMS, 256), _f32),
                  jax.ShapeDtypeStruct((NC, N_ATOMS, 128), _f32)],
        mesh=mesh,
        scratch_types=[pltpu.VMEM((1, 128), _i32),
                       pltpu.VMEM((128, 128), _f32),
                       pltpu.VMEM_SHARED((N_ATOMS, 128), _f32)],
    )
    return f(g, src3d, zeros)


ABLK = 400


def _ps_body(c_ref, pa_ref, pb_ref, p01_ref, p2_ref, r_ref, t_ref, o_ref):
    c2 = pa_ref[0] + pb_ref[0]
    d = _mm(c_ref[...], p01_ref[...]) + _mm(c2, p2_ref[...])
    rsel = r_ref[...]
    tsel = t_ref[...]
    plane = 0
    for l in range(LMAX + 1):
        m = 2 * l + 1
        scale = float(1.0 / np.sqrt(2 * l + 1))
        acc = None
        for mm in range(m):
            a = d[:, (plane + mm) * NPS:(plane + mm + 1) * NPS]
            x1 = _mm(a, rsel)
            x2 = _mm(a, tsel)
            acc = x1 * x2 if acc is None else acc + x1 * x2
        o_ref[:, l * NPS * NPS:(l + 1) * NPS * NPS] = acc * scale
        plane += m


def _tc_ps(coeff256, part):
    nblk = N_ATOMS // ABLK
    p01 = jnp.asarray(_PMAT[:256, :])
    p2 = jnp.asarray(_PMAT[256:, :])
    rsel = jnp.asarray(_RSEL)
    tsel = jnp.asarray(_TSEL)
    return pl.pallas_call(
        _ps_body,
        grid=(nblk,),
        in_specs=[
            pl.BlockSpec((ABLK, 256), lambda i: (i, 0)),
            pl.BlockSpec((1, ABLK, 128), lambda i: (0, i, 0)),
            pl.BlockSpec((1, ABLK, 128), lambda i: (1, i, 0)),
            pl.BlockSpec(p01.shape, lambda i: (0, 0)),
            pl.BlockSpec(p2.shape, lambda i: (0, 0)),
            pl.BlockSpec(rsel.shape, lambda i: (0, 0)),
            pl.BlockSpec(tsel.shape, lambda i: (0, 0)),
        ],
        out_specs=pl.BlockSpec((ABLK, OUT_W), lambda i: (i, 0)),
        out_shape=jax.ShapeDtypeStruct((N_ATOMS, OUT_W), _f32),
    )(coeff256, part, part, p01, p2, rsel, tsel)


def kernel(positions, species, pairs, W1, b1, W2, alch):
    wqt = jnp.take(alch, species, axis=0)
    zero_col = jnp.zeros((N_ATOMS, 1), _f32)
    recf = jnp.concatenate([positions, zero_col, wqt], axis=1).reshape(-1)
    src3d = pairs[0].reshape(ROWS, 1, 128)
    dst3d = pairs[1].reshape(ROWS, 1, 128)

    rsrc_f, rdst_f = _sc_gather(recf, src3d, dst3d)
    rsrc = rsrc_f.reshape(E_EDGES, 4)
    rdst = rdst_f.reshape(E_EDGES, 8)
    g = _tc_edges(rsrc, rdst, W1, b1.reshape(1, H), W2)
    zeros = jnp.zeros((ZCH, 128), _f32)
    coeff256, part = _sc_scatter(g, src3d, zeros)
    return _tc_ps(coeff256, part)


# speedup vs baseline: 43.4334x; 1.3130x over previous
import numpy as np

import jax
import jax.numpy as jnp
from jax import lax
from jax.experimental import pallas as pl
from jax.experimental.pallas import tpu as pltpu
from jax.experimental.pallas import tpu_sc as plsc

N_ATOMS = 10000
E_EDGES = 160000
Q = 4
NMAX = 6
LMAX = 3
RC = 5.0
H = 32
MOFF = (0, 1, 4, 9)
F96 = 96
FTOT = Q * F96
NPS = 24
OUT_W = (LMAX + 1) * NPS * NPS

NC, NS = 2, 16
NW = NC * NS
ROWS = E_EDGES // 128
ROWS_A = ROWS // NW
EXTRA_A = ROWS - ROWS_A * NW
ROWS_C = ROWS // NS
EXTRA_C = ROWS - ROWS_C * NS
ZCH = 624
HROWS = ROWS // NC

_f32 = jnp.float32
_i32 = jnp.int32


def _np_consts():
    c0 = 0.28209479177387814
    c1 = 0.4886025119029199
    c2a, c2b, c2c = 1.0925484305920792, 0.31539156525252005, 0.5462742152960396
    c3a, c3b, c3c, c3d = (0.5900435899266435, 2.890611442640554,
                          0.4570457994644658, 0.3731763325901154)
    yc = np.zeros((20, 16), np.float32)
    yc[0, 0] = c0
    yc[2, 1] = c1
    yc[3, 2] = c1
    yc[1, 3] = c1
    yc[4, 4] = c2a
    yc[5, 5] = c2a
    yc[9, 6] = 3.0 * c2b
    yc[0, 6] = -c2b
    yc[6, 7] = c2a
    yc[7, 8] = c2c
    yc[8, 8] = -c2c
    yc[10, 9] = 3.0 * c3a
    yc[11, 9] = -c3a
    yc[12, 10] = c3b
    yc[13, 11] = 5.0 * c3c
    yc[2, 11] = -c3c
    yc[14, 12] = 5.0 * c3d
    yc[3, 12] = -3.0 * c3d
    yc[15, 13] = 5.0 * c3c
    yc[1, 13] = -c3c
    yc[16, 14] = 1.445305721320277
    yc[17, 14] = -1.445305721320277
    yc[18, 15] = c3a
    yc[19, 15] = -3.0 * c3a

    sa = np.zeros((Q, F96), np.float32)
    sb = np.zeros((NPS, F96), np.float32)
    for q in range(Q):
        for ln in range(NPS):
            sa[q, q * NPS + ln] = 1.0
            sb[ln, q * NPS + ln] = 1.0
    snm = np.zeros((F96, FTOT), np.float32)
    sm = np.zeros((16, FTOT), np.float32)
    for q in range(Q):
        for l in range(LMAX + 1):
            m = 2 * l + 1
            for n in range(NMAX):
                for mm in range(m):
                    col = q * F96 + 6 * MOFF[l] + n * m + mm
                    snm[q * NPS + l * NMAX + n, col] = 1.0
                    sm[MOFF[l] + mm, col] = 1.0
    pmat = np.zeros((FTOT, FTOT), np.float32)
    plane = 0
    for l in range(LMAX + 1):
        m = 2 * l + 1
        for mm in range(m):
            for q in range(Q):
                for n in range(NMAX):
                    col = q * F96 + 6 * MOFF[l] + n * m + mm
                    pmat[col, plane * NPS + q * NMAX + n] = 1.0
            plane += 1
    rsel = np.zeros((NPS, NPS * NPS), np.float32)
    tsel = np.zeros((NPS, NPS * NPS), np.float32)
    for a in range(NPS):
        for b in range(NPS):
            rsel[a, a * NPS + b] = 1.0
            tsel[b, a * NPS + b] = 1.0
    return yc, sa, sb, snm, sm, pmat, rsel, tsel


_YC, _SA, _SB, _SNM, _SM, _PMAT, _RSEL, _TSEL = _np_consts()


def _mm1(a, b):
    return lax.dot_general(a, b, (((1,), (0,)), ((), ())),
                           preferred_element_type=_f32)


def _mm(a, b):
    ah = a.astype(jnp.bfloat16).astype(_f32)
    al = a - ah
    bh = b.astype(jnp.bfloat16).astype(_f32)
    bl = b - bh
    return (_mm1(ah, bh) + _mm1(al, bh)) + _mm1(ah, bl)


def _gather_body(recf, src3d, dst3d, rsrc, rdst,
                 table, sidx, didx, srow, drow):
    wid = lax.axis_index("s") * NC + lax.axis_index("c")
    pltpu.sync_copy(recf, table)
    lanes = lax.iota(_i32, 16)

    def do_row(row):
        pltpu.sync_copy(src3d.at[row], sidx)
        pltpu.sync_copy(dst3d.at[row], didx)
        for k in range(8):
            s16 = sidx[0, pl.ds(k * 16, 16)] * 8
            d16 = didx[0, pl.ds(k * 16, 16)] * 8
            sl4 = (lanes + k * 16) * 4
            dl8 = (lanes + k * 16) * 8
            for comp in range(3):
                vs = plsc.load_gather(table, [s16 + comp])
                plsc.store_scatter(srow, [sl4 + comp], vs)
            for comp in (0, 1, 2, 4, 5, 6, 7):
                vd = plsc.load_gather(table, [d16 + comp])
                plsc.store_scatter(drow, [dl8 + comp], vd)
        pltpu.sync_copy(srow, rsrc.at[pl.ds(row * 512, 512)])
        pltpu.sync_copy(drow, rdst.at[pl.ds(row * 1024, 1024)])

    def body(i, _):
        do_row(wid * ROWS_A + i)
        return 0

    lax.fori_loop(0, ROWS_A, body, 0)

    @pl.when(wid < EXTRA_A)
    def _():
        do_row(NW * ROWS_A + wid)


def _sc_gather(recf, src3d, dst3d):
    mesh = plsc.VectorSubcoreMesh(core_axis_name="c", subcore_axis_name="s")
    f = pl.kernel(
        _gather_body,
        out_type=[jax.ShapeDtypeStruct((E_EDGES * 4,), _f32),
                  jax.ShapeDtypeStruct((E_EDGES * 8,), _f32)],
        mesh=mesh,
        scratch_types=[pltpu.VMEM((N_ATOMS * 8,), _f32),
                       pltpu.VMEM((1, 128), _i32),
                       pltpu.VMEM((1, 128), _i32),
                       pltpu.VMEM((512,), _f32),
                       pltpu.VMEM((1024,), _f32)],
        compiler_params=pltpu.CompilerParams(needs_layout_passes=False),
    )
    return f(recf, src3d, dst3d)


EBLK = 640
IBLK = EBLK // 128


def _edge_body(rsrc_ref, rdst_ref, w1_ref, b1_ref, w2_ref,
               yc_ref, sa_ref, sb_ref, snm_ref, sm_ref, g_ref):
    rs = rsrc_ref[...]
    rd = rdst_ref[...]
    rx = rd[:, 0:1] - rs[:, 0:1]
    ry = rd[:, 1:2] - rs[:, 1:2]
    rz = rd[:, 2:3] - rs[:, 2:3]
    d2 = rx * rx + ry * ry + rz * rz + 1e-12
    dist = jnp.sqrt(d2)
    inv = 1.0 / dist
    fc = 0.5 * (jnp.cos(jnp.pi * jnp.minimum(dist, RC) / RC) + 1.0)
    fc = jnp.where(dist < RC, fc, 0.0)

    h = jnp.tanh(dist * w1_ref[...] + b1_ref[...])
    rn = _mm(h, w2_ref[...]) * fc

    x = rx * inv
    y = ry * inv
    z = rz * inv
    xx, yy, zz = x * x, y * y, z * z
    xy, yz, xz = x * y, y * z, x * z
    mono = (jnp.ones_like(x), x, y, z, xy, yz, xz, xx, yy, zz,
            xx * y, yy * y, xy * z, yz * z, zz * z, x * zz,
            xx * z, yy * z, xx * x, x * yy)
    yc = yc_ref[...]
    ysph = mono[0] * yc[0:1, :]
    for t in range(1, 20):
        ysph = ysph + mono[t] * yc[t:t + 1, :]

    wq = rd[:, 4:8]
    rnq = _mm(wq, sa_ref[...]) * _mm(rn, sb_ref[...])
    g = _mm(rnq, snm_ref[...]) * _mm(ysph, sm_ref[...])
    g_ref[...] = g


def _tc_edges(rsrc, rdst, W1, b1, W2):
    nblk = E_EDGES // EBLK
    consts = [jnp.asarray(a) for a in (_YC, _SA, _SB, _SNM, _SM)]
    cspecs = [pl.BlockSpec(a.shape, lambda i: (0, 0)) for a in consts]
    return pl.pallas_call(
        _edge_body,
        grid=(nblk,),
        in_specs=[
            pl.BlockSpec((EBLK, 4), lambda i: (i, 0)),
            pl.BlockSpec((EBLK, 8), lambda i: (i, 0)),
            pl.BlockSpec((1, H), lambda i: (0, 0)),
            pl.BlockSpec((1, H), lambda i: (0, 0)),
            pl.BlockSpec((H, NPS), lambda i: (0, 0)),
        ] + cspecs,
        out_specs=pl.BlockSpec((EBLK, FTOT), lambda i: (i, 0)),
        out_shape=jax.ShapeDtypeStruct((E_EDGES, FTOT), _f32),
    )(rsrc, rdst, W1, b1, W2, *consts)


def _scatter_body(g_hbm, src3d, zeros_hbm, coeff_hbm, part_hbm,
                  idxrow, gbuf, acc):
    c = lax.axis_index("c")
    t = lax.axis_index("s")

    def zero_acc():
        pltpu.sync_copy(zeros_hbm, acc.at[pl.ds(t * ZCH, ZCH)])

        @pl.when(t == 0)
        def _():
            pltpu.sync_copy(zeros_hbm.at[pl.ds(0, N_ATOMS - NS * ZCH)],
                            acc.at[pl.ds(NS * ZCH, N_ATOMS - NS * ZCH)])

    def do_row(row, col):
        pltpu.sync_copy(src3d.at[row], idxrow)
        pltpu.sync_copy(g_hbm.at[pl.ds(row * 128, 128), pl.ds(col, 128)],
                        gbuf)
        pltpu.sync_copy(gbuf, acc.at[idxrow.at[0]], add=True)

    zero_acc()
    plsc.subcore_barrier()

    def body1(i, _):
        do_row(t * ROWS_C + i, c * 128)
        return 0

    lax.fori_loop(0, ROWS_C, body1, 0)

    @pl.when(t < EXTRA_C)
    def _():
        do_row(NS * ROWS_C + t, c * 128)

    plsc.subcore_barrier()
    pltpu.sync_copy(acc.at[pl.ds(t * ZCH, ZCH)],
                    coeff_hbm.at[pl.ds(t * ZCH, ZCH), pl.ds(c * 128, 128)])

    @pl.when(t == 0)
    def _():
        pltpu.sync_copy(
            acc.at[pl.ds(NS * ZCH, N_ATOMS - NS * ZCH)],
            coeff_hbm.at[pl.ds(NS * ZCH, N_ATOMS - NS * ZCH),
                         pl.ds(c * 128, 128)])

    plsc.subcore_barrier()
    zero_acc()
    plsc.subcore_barrier()
    hrows = HROWS // NS

    def body2(i, _):
        do_row(c * HROWS + t * hrows + i, 2 * 128)
        return 0

    lax.fori_loop(0, hrows, body2, 0)

    @pl.when(t < HROWS - NS * hrows)
    def _():
        do_row(c * HROWS + NS * hrows + t, 2 * 128)

    plsc.subcore_barrier()
    pltpu.sync_copy(acc.at[pl.ds(t * ZCH, ZCH)],
                    part_hbm.at[c, pl.ds(t * ZCH, ZCH)])

    @pl.when(t == 0)
    def _():
        pltpu.sync_copy(acc.at[pl.ds(NS * ZCH, N_ATOMS - NS * ZCH)],
                        part_hbm.at[c, pl.ds(NS * ZCH, N_ATOMS - NS * ZCH)])


def _sc_scatter(g, src3d, zeros):
    mesh = plsc.VectorSubcoreMesh(core_axis_name="c", subcore_axis_name="s")
    f = pl.kernel(
        _scatter_body,
        out_type=[jax.ShapeDtypeStruct((N_ATOMS, 256), _f32),
                  jax.ShapeDtypeStruct((NC, N_ATOMS, 128), _f32)],
        mesh=mesh,
        scratch_types=[pltpu.VMEM((1, 128), _i32),
                       pltpu.VMEM((128, 128), _f32),
                       pltpu.VMEM_SHARED((N_ATOMS, 128), _f32)],
    )
    return f(g, src3d, zeros)


ABLK = 400


def _ps_body(c_ref, pa_ref, pb_ref, p01_ref, p2_ref, r_ref, t_ref, o_ref):
    c2 = pa_ref[0] + pb_ref[0]
    d = _mm(c_ref[...], p01_ref[...]) + _mm(c2, p2_ref[...])
    rsel = r_ref[...]
    tsel = t_ref[...]
    plane = 0
    for l in range(LMAX + 1):
        m = 2 * l + 1
        scale = float(1.0 / np.sqrt(2 * l + 1))
        acc = None
        for mm in range(m):
            a = d[:, (plane + mm) * NPS:(plane + mm + 1) * NPS]
            x1 = _mm(a, rsel)
            x2 = _mm(a, tsel)
            acc = x1 * x2 if acc is None else acc + x1 * x2
        o_ref[:, l * NPS * NPS:(l + 1) * NPS * NPS] = acc * scale
        plane += m


def _tc_ps(coeff256, part):
    nblk = N_ATOMS // ABLK
    p01 = jnp.asarray(_PMAT[:256, :])
    p2 = jnp.asarray(_PMAT[256:, :])
    rsel = jnp.asarray(_RSEL)
    tsel = jnp.asarray(_TSEL)
    return pl.pallas_call(
        _ps_body,
        grid=(nblk,),
        in_specs=[
            pl.BlockSpec((ABLK, 256), lambda i: (i, 0)),
            pl.BlockSpec((1, ABLK, 128), lambda i: (0, i, 0)),
            pl.BlockSpec((1, ABLK, 128), lambda i: (1, i, 0)),
            pl.BlockSpec(p01.shape, lambda i: (0, 0)),
            pl.BlockSpec(p2.shape, lambda i: (0, 0)),
            pl.BlockSpec(rsel.shape, lambda i: (0, 0)),
            pl.BlockSpec(tsel.shape, lambda i: (0, 0)),
        ],
        out_specs=pl.BlockSpec((ABLK, OUT_W), lambda i: (i, 0)),
        out_shape=jax.ShapeDtypeStruct((N_ATOMS, OUT_W), _f32),
    )(coeff256, part, part, p01, p2, rsel, tsel)


def kernel(positions, species, pairs, W1, b1, W2, alch):
    wqt = jnp.take(alch, species, axis=0)
    zero_col = jnp.zeros((N_ATOMS, 1), _f32)
    recf = jnp.concatenate([positions, zero_col, wqt], axis=1).reshape(-1)
    src3d = pairs[0].reshape(ROWS, 1, 128)
    dst3d = pairs[1].reshape(ROWS, 1, 128)

    rsrc_f, rdst_f = _sc_gather(recf, src3d, dst3d)
    rsrc = rsrc_f.reshape(E_EDGES, 4)
    rdst = rdst_f.reshape(E_EDGES, 8)
    g = _tc_edges(rsrc, rdst, W1, b1.reshape(1, H), W2)
    zeros = jnp.zeros((ZCH, 128), _f32)
    coeff256, part = _sc_scatter(g, src3d, zeros)
    return _tc_ps(coeff256, part)


# speedup vs baseline: 106.5879x; 2.4541x over previous
import numpy as np

import jax
import jax.numpy as jnp
from jax import lax
from jax.experimental import pallas as pl
from jax.experimental.pallas import tpu as pltpu
from jax.experimental.pallas import tpu_sc as plsc

N_ATOMS = 10000
E_EDGES = 160000
Q = 4
NMAX = 6
LMAX = 3
RC = 5.0
H = 32
MOFF = (0, 1, 4, 9)
F96 = 96
FTOT = Q * F96
NPS = 24
OUT_W = (LMAX + 1) * NPS * NPS

NC, NS = 2, 16
NW = NC * NS
ROWS = E_EDGES // 128
ROWS_A = ROWS // NW
EXTRA_A = ROWS - ROWS_A * NW
ROWS_C = ROWS // NS
EXTRA_C = ROWS - ROWS_C * NS
ZCH = 624
HROWS = ROWS // NC

_f32 = jnp.float32
_i32 = jnp.int32


def _np_consts():
    c0 = 0.28209479177387814
    c1 = 0.4886025119029199
    c2a, c2b, c2c = 1.0925484305920792, 0.31539156525252005, 0.5462742152960396
    c3a, c3b, c3c, c3d = (0.5900435899266435, 2.890611442640554,
                          0.4570457994644658, 0.3731763325901154)
    yc = np.zeros((20, 16), np.float32)
    yc[0, 0] = c0
    yc[2, 1] = c1
    yc[3, 2] = c1
    yc[1, 3] = c1
    yc[4, 4] = c2a
    yc[5, 5] = c2a
    yc[9, 6] = 3.0 * c2b
    yc[0, 6] = -c2b
    yc[6, 7] = c2a
    yc[7, 8] = c2c
    yc[8, 8] = -c2c
    yc[10, 9] = 3.0 * c3a
    yc[11, 9] = -c3a
    yc[12, 10] = c3b
    yc[13, 11] = 5.0 * c3c
    yc[2, 11] = -c3c
    yc[14, 12] = 5.0 * c3d
    yc[3, 12] = -3.0 * c3d
    yc[15, 13] = 5.0 * c3c
    yc[1, 13] = -c3c
    yc[16, 14] = 1.445305721320277
    yc[17, 14] = -1.445305721320277
    yc[18, 15] = c3a
    yc[19, 15] = -3.0 * c3a

    sa = np.zeros((48, F96), np.float32)
    sb = np.zeros((48, F96), np.float32)
    for q in range(Q):
        for ln in range(NPS):
            sa[4 + q, q * NPS + ln] = 1.0
            sb[8 + ln, q * NPS + ln] = 1.0
    snm = np.zeros((F96, FTOT), np.float32)
    sm = np.zeros((48, FTOT), np.float32)
    for q in range(Q):
        for l in range(LMAX + 1):
            m = 2 * l + 1
            for n in range(NMAX):
                for mm in range(m):
                    col = q * F96 + 6 * MOFF[l] + n * m + mm
                    snm[q * NPS + l * NMAX + n, col] = 1.0
                    sm[32 + MOFF[l] + mm, col] = 1.0
    pmat = np.zeros((FTOT, FTOT), np.float32)
    plane = 0
    for l in range(LMAX + 1):
        m = 2 * l + 1
        for mm in range(m):
            for q in range(Q):
                for n in range(NMAX):
                    col = q * F96 + 6 * MOFF[l] + n * m + mm
                    pmat[col, plane * NPS + q * NMAX + n] = 1.0
            plane += 1
    rsel = np.zeros((NPS, NPS * NPS), np.float32)
    tsel = np.zeros((NPS, NPS * NPS), np.float32)
    for a in range(NPS):
        for b in range(NPS):
            rsel[a, a * NPS + b] = 1.0
            tsel[b, a * NPS + b] = 1.0
    return yc, sa, sb, snm, sm, pmat, rsel, tsel


_YC, _SA, _SB, _SNM, _SM, _PMAT, _RSEL, _TSEL = _np_consts()


def _mm(a, b):
    return lax.dot_general(a, b, (((1,), (0,)), ((), ())),
                           preferred_element_type=_f32)


def _gather_body(recf, src3d, dst3d, redge,
                 table, sidx, didx, drow):
    wid = lax.axis_index("s") * NC + lax.axis_index("c")
    pltpu.sync_copy(recf, table)
    lanes = lax.iota(_i32, 16)

    def do_row(row):
        pltpu.sync_copy(src3d.at[row], sidx)
        pltpu.sync_copy(dst3d.at[row], didx)
        for k in range(8):
            s16 = sidx[0, pl.ds(k * 16, 16)] * 8
            d16 = didx[0, pl.ds(k * 16, 16)] * 8
            dl8 = (lanes + k * 16) * 8
            for comp in range(3):
                vs = plsc.load_gather(table, [s16 + (5 + comp)])
                vd = plsc.load_gather(table, [d16 + (5 + comp)])
                plsc.store_scatter(drow, [dl8 + comp], vd - vs)
            for comp in range(4):
                vd = plsc.load_gather(table, [d16 + comp])
                plsc.store_scatter(drow, [dl8 + 4 + comp], vd)
        pltpu.sync_copy(drow, redge.at[pl.ds(row * 1024, 1024)])

    def body(i, _):
        do_row(wid * ROWS_A + i)
        return 0

    lax.fori_loop(0, ROWS_A, body, 0)

    @pl.when(wid < EXTRA_A)
    def _():
        do_row(NW * ROWS_A + wid)


def _sc_gather(recf, src3d, dst3d):
    mesh = plsc.VectorSubcoreMesh(core_axis_name="c", subcore_axis_name="s")
    f = pl.kernel(
        _gather_body,
        out_type=jax.ShapeDtypeStruct((E_EDGES * 8,), _f32),
        mesh=mesh,
        scratch_types=[pltpu.VMEM((N_ATOMS * 8,), _f32),
                       pltpu.VMEM((1, 128), _i32),
                       pltpu.VMEM((1, 128), _i32),
                       pltpu.VMEM((1024,), _f32)],
        compiler_params=pltpu.CompilerParams(needs_layout_passes=False),
    )
    return f(recf, src3d, dst3d)


EBLK = 1280


def _edge_body(red_ref, w1_ref, b1_ref, w2_ref,
               yct_ref, sa_ref, sb_ref, snm_ref, sm_ref, g_ref):
    rdt = lax.transpose(red_ref[...], (1, 0))
    rx = rdt[0:1, :]
    ry = rdt[1:2, :]
    rz = rdt[2:3, :]
    d2 = rx * rx + ry * ry + rz * rz + 1e-12
    dist = jnp.sqrt(d2)
    inv = 1.0 / dist
    fc = 0.5 * (jnp.cos(jnp.pi * jnp.minimum(dist, RC) / RC) + 1.0)
    fc = jnp.where(dist < RC, fc, 0.0)

    h = jnp.tanh(w1_ref[...] * dist + b1_ref[...])
    rn = _mm(w2_ref[...], h) * fc

    x = rx * inv
    y = ry * inv
    z = rz * inv
    xx, yy, zz = x * x, y * y, z * z
    xy, yz, xz = x * y, y * z, x * z
    mono = (jnp.ones_like(x), x, y, z, xy, yz, xz, xx, yy, zz,
            xx * y, yy * y, xy * z, yz * z, zz * z, x * zz,
            xx * z, yy * z, xx * x, x * yy)
    yct = yct_ref[...]
    ysph = yct[:, 0:1] * mono[0]
    for t in range(1, 20):
        ysph = ysph + yct[:, t:t + 1] * mono[t]

    f = jnp.concatenate([rdt, rn, ysph], axis=0)
    fe = lax.transpose(f, (1, 0))
    rnq = _mm(fe, sa_ref[...]) * _mm(fe, sb_ref[...])
    g = _mm(rnq, snm_ref[...]) * _mm(fe, sm_ref[...])
    g_ref[...] = g


def _tc_edges(redge, W1, b1, W2):
    nblk = E_EDGES // EBLK
    consts = [jnp.asarray(a) for a in (_YC.T, _SA, _SB, _SNM, _SM)]
    cspecs = [pl.BlockSpec(a.shape, lambda i: (0, 0)) for a in consts]
    return pl.pallas_call(
        _edge_body,
        grid=(nblk,),
        in_specs=[
            pl.BlockSpec((EBLK, 8), lambda i: (i, 0)),
            pl.BlockSpec((H, 1), lambda i: (0, 0)),
            pl.BlockSpec((H, 1), lambda i: (0, 0)),
            pl.BlockSpec((NPS, H), lambda i: (0, 0)),
        ] + cspecs,
        out_specs=pl.BlockSpec((EBLK, FTOT), lambda i: (i, 0)),
        out_shape=jax.ShapeDtypeStruct((E_EDGES, FTOT), _f32),
    )(redge, W1.reshape(H, 1), b1.reshape(H, 1), W2.T, *consts)


def _scatter_body(g_hbm, src3d, zeros_hbm, coeff_hbm, part_hbm,
                  idxrow, gbuf, acc):
    c = lax.axis_index("c")
    t = lax.axis_index("s")

    def zero_acc():
        pltpu.sync_copy(zeros_hbm, acc.at[pl.ds(t * ZCH, ZCH)])

        @pl.when(t == 0)
        def _():
            pltpu.sync_copy(zeros_hbm.at[pl.ds(0, N_ATOMS - NS * ZCH)],
                            acc.at[pl.ds(NS * ZCH, N_ATOMS - NS * ZCH)])

    def do_row(row, col):
        pltpu.sync_copy(src3d.at[row], idxrow)
        pltpu.sync_copy(g_hbm.at[pl.ds(row * 128, 128), pl.ds(col, 128)],
                        gbuf)
        pltpu.sync_copy(gbuf, acc.at[idxrow.at[0]], add=True)

    zero_acc()
    plsc.subcore_barrier()

    def body1(i, _):
        do_row(t * ROWS_C + i, c * 128)
        return 0

    lax.fori_loop(0, ROWS_C, body1, 0)

    @pl.when(t < EXTRA_C)
    def _():
        do_row(NS * ROWS_C + t, c * 128)

    plsc.subcore_barrier()
    pltpu.sync_copy(acc.at[pl.ds(t * ZCH, ZCH)],
                    coeff_hbm.at[pl.ds(t * ZCH, ZCH), pl.ds(c * 128, 128)])

    @pl.when(t == 0)
    def _():
        pltpu.sync_copy(
            acc.at[pl.ds(NS * ZCH, N_ATOMS - NS * ZCH)],
            coeff_hbm.at[pl.ds(NS * ZCH, N_ATOMS - NS * ZCH),
                         pl.ds(c * 128, 128)])

    plsc.subcore_barrier()
    zero_acc()
    plsc.subcore_barrier()
    hrows = HROWS // NS

    def body2(i, _):
        do_row(c * HROWS + t * hrows + i, 2 * 128)
        return 0

    lax.fori_loop(0, hrows, body2, 0)

    @pl.when(t < HROWS - NS * hrows)
    def _():
        do_row(c * HROWS + NS * hrows + t, 2 * 128)

    plsc.subcore_barrier()
    pltpu.sync_copy(acc.at[pl.ds(t * ZCH, ZCH)],
                    part_hbm.at[c, pl.ds(t * ZCH, ZCH)])

    @pl.when(t == 0)
    def _():
        pltpu.sync_copy(acc.at[pl.ds(NS * ZCH, N_ATOMS - NS * ZCH)],
                        part_hbm.at[c, pl.ds(NS * ZCH, N_ATOMS - NS * ZCH)])


def _sc_scatter(g, src3d, zeros):
    mesh = plsc.VectorSubcoreMesh(core_axis_name="c", subcore_axis_name="s")
    f = pl.kernel(
        _scatter_body,
        out_type=[jax.ShapeDtypeStruct((N_ATOMS, 256), _f32),
                  jax.ShapeDtypeStruct((NC, N_ATOMS, 128), _f32)],
        mesh=mesh,
        scratch_types=[pltpu.VMEM((1, 128), _i32),
                       pltpu.VMEM((128, 128), _f32),
                       pltpu.VMEM_SHARED((N_ATOMS, 128), _f32)],
    )
    return f(g, src3d, zeros)


ABLK = 400


def _ps_body(c_ref, pa_ref, pb_ref, p01_ref, p2_ref, r_ref, t_ref, o_ref):
    c2 = pa_ref[0] + pb_ref[0]
    d = _mm(c_ref[...], p01_ref[...]) + _mm(c2, p2_ref[...])
    rsel = r_ref[...]
    tsel = t_ref[...]
    plane = 0
    for l in range(LMAX + 1):
        m = 2 * l + 1
        scale = float(1.0 / np.sqrt(2 * l + 1))
        acc = None
        for mm in range(m):
            a = d[:, (plane + mm) * NPS:(plane + mm + 1) * NPS]
            x1 = _mm(a, rsel)
            x2 = _mm(a, tsel)
            acc = x1 * x2 if acc is None else acc + x1 * x2
        o_ref[:, l * NPS * NPS:(l + 1) * NPS * NPS] = acc * scale
        plane += m


def _tc_ps(coeff256, part):
    nblk = N_ATOMS // ABLK
    p01 = jnp.asarray(_PMAT[:256, :])
    p2 = jnp.asarray(_PMAT[256:, :])
    rsel = jnp.asarray(_RSEL)
    tsel = jnp.asarray(_TSEL)
    return pl.pallas_call(
        _ps_body,
        grid=(nblk,),
        in_specs=[
            pl.BlockSpec((ABLK, 256), lambda i: (i, 0)),
            pl.BlockSpec((1, ABLK, 128), lambda i: (0, i, 0)),
            pl.BlockSpec((1, ABLK, 128), lambda i: (1, i, 0)),
            pl.BlockSpec(p01.shape, lambda i: (0, 0)),
            pl.BlockSpec(p2.shape, lambda i: (0, 0)),
            pl.BlockSpec(rsel.shape, lambda i: (0, 0)),
            pl.BlockSpec(tsel.shape, lambda i: (0, 0)),
        ],
        out_specs=pl.BlockSpec((ABLK, OUT_W), lambda i: (i, 0)),
        out_shape=jax.ShapeDtypeStruct((N_ATOMS, OUT_W), _f32),
    )(coeff256, part, part, p01, p2, rsel, tsel)


def kernel(positions, species, pairs, W1, b1, W2, alch):
    wqt = jnp.take(alch, species, axis=0)
    zero_col = jnp.zeros((N_ATOMS, 1), _f32)
    recf = jnp.concatenate([wqt, zero_col, positions], axis=1).reshape(-1)
    src3d = pairs[0].reshape(ROWS, 1, 128)
    dst3d = pairs[1].reshape(ROWS, 1, 128)

    redge = _sc_gather(recf, src3d, dst3d).reshape(E_EDGES, 8)
    g = _tc_edges(redge, W1, b1, W2)
    zeros = jnp.zeros((ZCH, 128), _f32)
    coeff256, part = _sc_scatter(g, src3d, zeros)
    return _tc_ps(coeff256, part)


# speedup vs baseline: 119.4822x; 1.1210x over previous
import numpy as np

import jax
import jax.numpy as jnp
from jax import lax
from jax.experimental import pallas as pl
from jax.experimental.pallas import tpu as pltpu
from jax.experimental.pallas import tpu_sc as plsc

N_ATOMS = 10000
E_EDGES = 160000
Q = 4
NMAX = 6
LMAX = 3
RC = 5.0
H = 32
MOFF = (0, 1, 4, 9)
F96 = 96
FTOT = Q * F96
NPS = 24
OUT_W = (LMAX + 1) * NPS * NPS

NC, NS = 2, 16
NW = NC * NS
ROWS = E_EDGES // 128
ROWS_A = ROWS // NW
EXTRA_A = ROWS - ROWS_A * NW
ROWS_C = ROWS // NS
EXTRA_C = ROWS - ROWS_C * NS
ZCH = 624
HROWS = ROWS // NC

_f32 = jnp.float32
_i32 = jnp.int32


def _np_consts():
    c0 = 0.28209479177387814
    c1 = 0.4886025119029199
    c2a, c2b, c2c = 1.0925484305920792, 0.31539156525252005, 0.5462742152960396
    c3a, c3b, c3c, c3d = (0.5900435899266435, 2.890611442640554,
                          0.4570457994644658, 0.3731763325901154)
    yc = np.zeros((20, 16), np.float32)
    yc[0, 0] = c0
    yc[2, 1] = c1
    yc[3, 2] = c1
    yc[1, 3] = c1
    yc[4, 4] = c2a
    yc[5, 5] = c2a
    yc[9, 6] = 3.0 * c2b
    yc[0, 6] = -c2b
    yc[6, 7] = c2a
    yc[7, 8] = c2c
    yc[8, 8] = -c2c
    yc[10, 9] = 3.0 * c3a
    yc[11, 9] = -c3a
    yc[12, 10] = c3b
    yc[13, 11] = 5.0 * c3c
    yc[2, 11] = -c3c
    yc[14, 12] = 5.0 * c3d
    yc[3, 12] = -3.0 * c3d
    yc[15, 13] = 5.0 * c3c
    yc[1, 13] = -c3c
    yc[16, 14] = 1.445305721320277
    yc[17, 14] = -1.445305721320277
    yc[18, 15] = c3a
    yc[19, 15] = -3.0 * c3a

    sa = np.zeros((48, F96), np.float32)
    sb = np.zeros((48, F96), np.float32)
    for q in range(Q):
        for ln in range(NPS):
            sa[4 + q, q * NPS + ln] = 1.0
            sb[8 + ln, q * NPS + ln] = 1.0
    snm = np.zeros((F96, FTOT), np.float32)
    sm = np.zeros((48, FTOT), np.float32)
    for q in range(Q):
        for l in range(LMAX + 1):
            m = 2 * l + 1
            for n in range(NMAX):
                for mm in range(m):
                    col = q * F96 + 6 * MOFF[l] + n * m + mm
                    snm[q * NPS + l * NMAX + n, col] = 1.0
                    sm[32 + MOFF[l] + mm, col] = 1.0
    pmat = np.zeros((FTOT, FTOT), np.float32)
    plane = 0
    for l in range(LMAX + 1):
        m = 2 * l + 1
        for mm in range(m):
            for q in range(Q):
                for n in range(NMAX):
                    col = q * F96 + 6 * MOFF[l] + n * m + mm
                    pmat[col, plane * NPS + q * NMAX + n] = 1.0
            plane += 1
    rsel = np.zeros((NPS, NPS * NPS), np.float32)
    tsel = np.zeros((NPS, NPS * NPS), np.float32)
    for a in range(NPS):
        for b in range(NPS):
            rsel[a, a * NPS + b] = 1.0
            tsel[b, a * NPS + b] = 1.0
    return yc, sa, sb, snm, sm, pmat, rsel, tsel


_YC, _SA, _SB, _SNM, _SM, _PMAT, _RSEL, _TSEL = _np_consts()


def _mm(a, b):
    return lax.dot_general(a, b, (((1,), (0,)), ((), ())),
                           preferred_element_type=_f32)


def _gather_body(recf, src3d, dst3d, redge,
                 table, sidx, didx, drow):
    wid = lax.axis_index("s") * NC + lax.axis_index("c")
    pltpu.sync_copy(recf, table)
    lanes = lax.iota(_i32, 16)

    def do_row(row):
        pltpu.sync_copy(src3d.at[row], sidx)
        pltpu.sync_copy(dst3d.at[row], didx)
        for k in range(8):
            s16 = sidx[0, pl.ds(k * 16, 16)] * 8
            d16 = didx[0, pl.ds(k * 16, 16)] * 8
            dl8 = (lanes + k * 16) * 8
            for comp in range(3):
                vs = plsc.load_gather(table, [s16 + (5 + comp)])
                vd = plsc.load_gather(table, [d16 + (5 + comp)])
                plsc.store_scatter(drow, [dl8 + comp], vd - vs)
            for comp in range(4):
                vd = plsc.load_gather(table, [d16 + comp])
                plsc.store_scatter(drow, [dl8 + 4 + comp], vd)
        pltpu.sync_copy(drow, redge.at[pl.ds(row * 1024, 1024)])

    def body(i, _):
        do_row(wid * ROWS_A + i)
        return 0

    lax.fori_loop(0, ROWS_A, body, 0)

    @pl.when(wid < EXTRA_A)
    def _():
        do_row(NW * ROWS_A + wid)


def _sc_gather(recf, src3d, dst3d):
    mesh = plsc.VectorSubcoreMesh(core_axis_name="c", subcore_axis_name="s")
    f = pl.kernel(
        _gather_body,
        out_type=jax.ShapeDtypeStruct((E_EDGES * 8,), _f32),
        mesh=mesh,
        scratch_types=[pltpu.VMEM((N_ATOMS * 8,), _f32),
                       pltpu.VMEM((1, 128), _i32),
                       pltpu.VMEM((1, 128), _i32),
                       pltpu.VMEM((1024,), _f32)],
        compiler_params=pltpu.CompilerParams(needs_layout_passes=False),
    )
    return f(recf, src3d, dst3d)


EBLK = 1280


def _edge_body(red_ref, w1_ref, b1_ref, w2_ref,
               yct_ref, sa_ref, sb_ref, snm_ref, sm_ref, g_ref):
    rdt = lax.transpose(red_ref[...], (1, 0))
    rx = rdt[0:1, :]
    ry = rdt[1:2, :]
    rz = rdt[2:3, :]
    d2 = rx * rx + ry * ry + rz * rz + 1e-12
    dist = jnp.sqrt(d2)
    inv = 1.0 / dist
    fc = 0.5 * (jnp.cos(jnp.pi * jnp.minimum(dist, RC) / RC) + 1.0)
    fc = jnp.where(dist < RC, fc, 0.0)

    h = jnp.tanh(w1_ref[...] * dist + b1_ref[...])
    rn = _mm(w2_ref[...], h) * fc

    x = rx * inv
    y = ry * inv
    z = rz * inv
    xx, yy, zz = x * x, y * y, z * z
    xy, yz, xz = x * y, y * z, x * z
    mono = (jnp.ones_like(x), x, y, z, xy, yz, xz, xx, yy, zz,
            xx * y, yy * y, xy * z, yz * z, zz * z, x * zz,
            xx * z, yy * z, xx * x, x * yy)
    yct = yct_ref[...]
    ysph = yct[:, 0:1] * mono[0]
    for t in range(1, 20):
        ysph = ysph + yct[:, t:t + 1] * mono[t]

    f = jnp.concatenate([rdt, rn, ysph], axis=0)
    fe = lax.transpose(f, (1, 0))
    rnq = _mm(fe, sa_ref[...]) * _mm(fe, sb_ref[...])
    g = _mm(rnq, snm_ref[...]) * _mm(fe, sm_ref[...])
    g_ref[...] = g


def _tc_edges(redge, W1, b1, W2):
    nblk = E_EDGES // EBLK
    consts = [jnp.asarray(a) for a in (_YC.T, _SA, _SB, _SNM, _SM)]
    cspecs = [pl.BlockSpec(a.shape, lambda i: (0, 0)) for a in consts]
    return pl.pallas_call(
        _edge_body,
        grid=(nblk,),
        in_specs=[
            pl.BlockSpec((EBLK, 8), lambda i: (i, 0)),
            pl.BlockSpec((H, 1), lambda i: (0, 0)),
            pl.BlockSpec((H, 1), lambda i: (0, 0)),
            pl.BlockSpec((NPS, H), lambda i: (0, 0)),
        ] + cspecs,
        out_specs=pl.BlockSpec((EBLK, FTOT), lambda i: (i, 0)),
        out_shape=jax.ShapeDtypeStruct((E_EDGES, FTOT), _f32),
    )(redge, W1.reshape(H, 1), b1.reshape(H, 1), W2.T, *consts)


def _scatter_body(g_hbm, src3d, zeros_hbm, coeff_hbm, part_hbm,
                  idxrow, idxbuf, gbuf2, sem0, sem1, acc):
    c = lax.axis_index("c")
    t = lax.axis_index("s")

    def zero_acc():
        pltpu.sync_copy(zeros_hbm, acc.at[pl.ds(t * ZCH, ZCH)])

        @pl.when(t == 0)
        def _():
            pltpu.sync_copy(zeros_hbm.at[pl.ds(0, N_ATOMS - NS * ZCH)],
                            acc.at[pl.ds(NS * ZCH, N_ATOMS - NS * ZCH)])

    def do_row(row, col):
        pltpu.sync_copy(src3d.at[row], idxrow)
        pltpu.sync_copy(g_hbm.at[pl.ds(row * 128, 128), pl.ds(col, 128)],
                        gbuf2.at[0])
        pltpu.sync_copy(gbuf2.at[0], acc.at[idxrow.at[0]], add=True)

    def run_rows(start, n, col):
        pltpu.sync_copy(src3d.at[pl.ds(start, n)], idxbuf.at[pl.ds(0, n)])

        def pair(j, _):
            r0 = start + 2 * j
            c0 = pltpu.async_copy(
                g_hbm.at[pl.ds(r0 * 128, 128), pl.ds(col, 128)],
                gbuf2.at[0], sem0)
            c1 = pltpu.async_copy(
                g_hbm.at[pl.ds((r0 + 1) * 128, 128), pl.ds(col, 128)],
                gbuf2.at[1], sem1)
            c0.wait()
            pltpu.sync_copy(gbuf2.at[0], acc.at[idxbuf.at[2 * j, 0]],
                            add=True)
            c1.wait()
            pltpu.sync_copy(gbuf2.at[1], acc.at[idxbuf.at[2 * j + 1, 0]],
                            add=True)
            return 0

        lax.fori_loop(0, n // 2, pair, 0)
        if n % 2:
            r = start + n - 1
            pltpu.sync_copy(
                g_hbm.at[pl.ds(r * 128, 128), pl.ds(col, 128)], gbuf2.at[0])
            pltpu.sync_copy(gbuf2.at[0], acc.at[idxbuf.at[n - 1, 0]],
                            add=True)

    zero_acc()
    plsc.subcore_barrier()
    run_rows(t * ROWS_C, ROWS_C, c * 128)

    @pl.when(t < EXTRA_C)
    def _():
        do_row(NS * ROWS_C + t, c * 128)

    plsc.subcore_barrier()
    pltpu.sync_copy(acc.at[pl.ds(t * ZCH, ZCH)],
                    coeff_hbm.at[pl.ds(t * ZCH, ZCH), pl.ds(c * 128, 128)])

    @pl.when(t == 0)
    def _():
        pltpu.sync_copy(
            acc.at[pl.ds(NS * ZCH, N_ATOMS - NS * ZCH)],
            coeff_hbm.at[pl.ds(NS * ZCH, N_ATOMS - NS * ZCH),
                         pl.ds(c * 128, 128)])

    plsc.subcore_barrier()
    zero_acc()
    plsc.subcore_barrier()
    hrows = HROWS // NS
    run_rows(c * HROWS + t * hrows, hrows, 2 * 128)

    @pl.when(t < HROWS - NS * hrows)
    def _():
        do_row(c * HROWS + NS * hrows + t, 2 * 128)

    plsc.subcore_barrier()
    pltpu.sync_copy(acc.at[pl.ds(t * ZCH, ZCH)],
                    part_hbm.at[c, pl.ds(t * ZCH, ZCH)])

    @pl.when(t == 0)
    def _():
        pltpu.sync_copy(acc.at[pl.ds(NS * ZCH, N_ATOMS - NS * ZCH)],
                        part_hbm.at[c, pl.ds(NS * ZCH, N_ATOMS - NS * ZCH)])


def _sc_scatter(g, src3d, zeros):
    mesh = plsc.VectorSubcoreMesh(core_axis_name="c", subcore_axis_name="s")
    f = pl.kernel(
        _scatter_body,
        out_type=[jax.ShapeDtypeStruct((N_ATOMS, 256), _f32),
                  jax.ShapeDtypeStruct((NC, N_ATOMS, 128), _f32)],
        mesh=mesh,
        scratch_types=[pltpu.VMEM((1, 128), _i32),
                       pltpu.VMEM((ROWS_C, 1, 128), _i32),
                       pltpu.VMEM((2, 128, 128), _f32),
                       pltpu.SemaphoreType.DMA,
                       pltpu.SemaphoreType.DMA,
                       pltpu.VMEM_SHARED((N_ATOMS, 128), _f32)],
    )
    return f(g, src3d, zeros)


ABLK = 400


def _ps_body(c_ref, pa_ref, pb_ref, p01_ref, p2_ref, r_ref, t_ref, o_ref):
    c2 = pa_ref[0] + pb_ref[0]
    d = _mm(c_ref[...], p01_ref[...]) + _mm(c2, p2_ref[...])
    rsel = r_ref[...]
    tsel = t_ref[...]
    plane = 0
    for l in range(LMAX + 1):
        m = 2 * l + 1
        scale = float(1.0 / np.sqrt(2 * l + 1))
        acc = None
        for mm in range(m):
            a = d[:, (plane + mm) * NPS:(plane + mm + 1) * NPS]
            x1 = _mm(a, rsel)
            x2 = _mm(a, tsel)
            acc = x1 * x2 if acc is None else acc + x1 * x2
        o_ref[:, l * NPS * NPS:(l + 1) * NPS * NPS] = acc * scale
        plane += m


def _tc_ps(coeff256, part):
    nblk = N_ATOMS // ABLK
    p01 = jnp.asarray(_PMAT[:256, :])
    p2 = jnp.asarray(_PMAT[256:, :])
    rsel = jnp.asarray(_RSEL)
    tsel = jnp.asarray(_TSEL)
    return pl.pallas_call(
        _ps_body,
        grid=(nblk,),
        in_specs=[
            pl.BlockSpec((ABLK, 256), lambda i: (i, 0)),
            pl.BlockSpec((1, ABLK, 128), lambda i: (0, i, 0)),
            pl.BlockSpec((1, ABLK, 128), lambda i: (1, i, 0)),
            pl.BlockSpec(p01.shape, lambda i: (0, 0)),
            pl.BlockSpec(p2.shape, lambda i: (0, 0)),
            pl.BlockSpec(rsel.shape, lambda i: (0, 0)),
            pl.BlockSpec(tsel.shape, lambda i: (0, 0)),
        ],
        out_specs=pl.BlockSpec((ABLK, OUT_W), lambda i: (i, 0)),
        out_shape=jax.ShapeDtypeStruct((N_ATOMS, OUT_W), _f32),
    )(coeff256, part, part, p01, p2, rsel, tsel)


def kernel(positions, species, pairs, W1, b1, W2, alch):
    wqt = jnp.take(alch, species, axis=0)
    zero_col = jnp.zeros((N_ATOMS, 1), _f32)
    recf = jnp.concatenate([wqt, zero_col, positions], axis=1).reshape(-1)
    src3d = pairs[0].reshape(ROWS, 1, 128)
    dst3d = pairs[1].reshape(ROWS, 1, 128)

    redge = _sc_gather(recf, src3d, dst3d).reshape(E_EDGES, 8)
    g = _tc_edges(redge, W1, b1, W2)
    zeros = jnp.zeros((ZCH, 128), _f32)
    coeff256, part = _sc_scatter(g, src3d, zeros)
    return _tc_ps(coeff256, part)


# speedup vs baseline: 125.7249x; 1.0522x over previous
import numpy as np

import jax
import jax.numpy as jnp
from jax import lax
from jax.experimental import pallas as pl
from jax.experimental.pallas import tpu as pltpu
from jax.experimental.pallas import tpu_sc as plsc

N_ATOMS = 10000
E_EDGES = 160000
Q = 4
NMAX = 6
LMAX = 3
RC = 5.0
H = 32
MOFF = (0, 1, 4, 9)
F96 = 96
FTOT = Q * F96
NPS = 24
OUT_W = (LMAX + 1) * NPS * NPS

NC, NS = 2, 16
NW = NC * NS
ROWS = E_EDGES // 128
ROWS_A = ROWS // NW
EXTRA_A = ROWS - ROWS_A * NW
ROWS_C = ROWS // NS
EXTRA_C = ROWS - ROWS_C * NS
ZCH = 624
HROWS = ROWS // NC

_f32 = jnp.float32
_i32 = jnp.int32


def _np_consts():
    c0 = 0.28209479177387814
    c1 = 0.4886025119029199
    c2a, c2b, c2c = 1.0925484305920792, 0.31539156525252005, 0.5462742152960396
    c3a, c3b, c3c, c3d = (0.5900435899266435, 2.890611442640554,
                          0.4570457994644658, 0.3731763325901154)
    yc = np.zeros((20, 16), np.float32)
    yc[0, 0] = c0
    yc[2, 1] = c1
    yc[3, 2] = c1
    yc[1, 3] = c1
    yc[4, 4] = c2a
    yc[5, 5] = c2a
    yc[9, 6] = 3.0 * c2b
    yc[0, 6] = -c2b
    yc[6, 7] = c2a
    yc[7, 8] = c2c
    yc[8, 8] = -c2c
    yc[10, 9] = 3.0 * c3a
    yc[11, 9] = -c3a
    yc[12, 10] = c3b
    yc[13, 11] = 5.0 * c3c
    yc[2, 11] = -c3c
    yc[14, 12] = 5.0 * c3d
    yc[3, 12] = -3.0 * c3d
    yc[15, 13] = 5.0 * c3c
    yc[1, 13] = -c3c
    yc[16, 14] = 1.445305721320277
    yc[17, 14] = -1.445305721320277
    yc[18, 15] = c3a
    yc[19, 15] = -3.0 * c3a

    sa = np.zeros((48, F96), np.float32)
    sb = np.zeros((48, F96), np.float32)
    for q in range(Q):
        for ln in range(NPS):
            sa[4 + q, q * NPS + ln] = 1.0
            sb[8 + ln, q * NPS + ln] = 1.0
    snm = np.zeros((F96, FTOT), np.float32)
    sm = np.zeros((48, FTOT), np.float32)
    for q in range(Q):
        for l in range(LMAX + 1):
            m = 2 * l + 1
            for n in range(NMAX):
                for mm in range(m):
                    col = q * F96 + 6 * MOFF[l] + n * m + mm
                    snm[q * NPS + l * NMAX + n, col] = 1.0
                    sm[32 + MOFF[l] + mm, col] = 1.0
    pmat = np.zeros((FTOT, FTOT), np.float32)
    plane = 0
    for l in range(LMAX + 1):
        m = 2 * l + 1
        for mm in range(m):
            for q in range(Q):
                for n in range(NMAX):
                    col = q * F96 + 6 * MOFF[l] + n * m + mm
                    pmat[col, plane * NPS + q * NMAX + n] = 1.0
            plane += 1
    rsel = np.zeros((NPS, NPS * NPS), np.float32)
    tsel = np.zeros((NPS, NPS * NPS), np.float32)
    for a in range(NPS):
        for b in range(NPS):
            rsel[a, a * NPS + b] = 1.0
            tsel[b, a * NPS + b] = 1.0
    return yc, sa, sb, snm, sm, pmat, rsel, tsel


_YC, _SA, _SB, _SNM, _SM, _PMAT, _RSEL, _TSEL = _np_consts()


def _mm(a, b):
    return lax.dot_general(a, b, (((1,), (0,)), ((), ())),
                           preferred_element_type=_f32)


def _gather_body(recf, src3d, dst3d, redge,
                 table, sidxb, didxb, drow2, wsem0, wsem1):
    wid = lax.axis_index("s") * NC + lax.axis_index("c")
    pltpu.sync_copy(recf, table)
    lanes = lax.iota(_i32, 16)
    start = wid * ROWS_A
    pltpu.sync_copy(src3d.at[pl.ds(start, ROWS_A)],
                    sidxb.at[pl.ds(0, ROWS_A)])
    pltpu.sync_copy(dst3d.at[pl.ds(start, ROWS_A)],
                    didxb.at[pl.ds(0, ROWS_A)])

    @pl.when(wid < EXTRA_A)
    def _():
        pltpu.sync_copy(src3d.at[pl.ds(NW * ROWS_A + wid, 1)],
                        sidxb.at[pl.ds(ROWS_A, 1)])
        pltpu.sync_copy(dst3d.at[pl.ds(NW * ROWS_A + wid, 1)],
                        didxb.at[pl.ds(ROWS_A, 1)])

    def build(j, slot):
        for k in range(8):
            s16 = sidxb[j, 0, pl.ds(k * 16, 16)] * 8
            d16 = didxb[j, 0, pl.ds(k * 16, 16)] * 8
            dl8 = slot * 1024 + (lanes + k * 16) * 8
            for comp in range(3):
                vs = plsc.load_gather(table, [s16 + (5 + comp)])
                vd = plsc.load_gather(table, [d16 + (5 + comp)])
                plsc.store_scatter(drow2, [dl8 + comp], vd - vs)
            for comp in range(4):
                vd = plsc.load_gather(table, [d16 + comp])
                plsc.store_scatter(drow2, [dl8 + 4 + comp], vd)

    def pair(i, _):
        r0 = start + 2 * i
        build(2 * i, 0)
        c0 = pltpu.async_copy(drow2.at[pl.ds(0, 1024)],
                              redge.at[pl.ds(r0 * 1024, 1024)], wsem0)
        build(2 * i + 1, 1)
        c1 = pltpu.async_copy(drow2.at[pl.ds(1024, 1024)],
                              redge.at[pl.ds((r0 + 1) * 1024, 1024)], wsem1)
        c0.wait()
        c1.wait()
        return 0

    lax.fori_loop(0, ROWS_A // 2, pair, 0)
    if ROWS_A % 2:
        build(ROWS_A - 1, 0)
        pltpu.sync_copy(drow2.at[pl.ds(0, 1024)],
                        redge.at[pl.ds((start + ROWS_A - 1) * 1024, 1024)])

    @pl.when(wid < EXTRA_A)
    def _():
        build(ROWS_A, 0)
        pltpu.sync_copy(
            drow2.at[pl.ds(0, 1024)],
            redge.at[pl.ds((NW * ROWS_A + wid) * 1024, 1024)])


def _sc_gather(recf, src3d, dst3d):
    mesh = plsc.VectorSubcoreMesh(core_axis_name="c", subcore_axis_name="s")
    f = pl.kernel(
        _gather_body,
        out_type=jax.ShapeDtypeStruct((E_EDGES * 8,), _f32),
        mesh=mesh,
        scratch_types=[pltpu.VMEM((N_ATOMS * 8,), _f32),
                       pltpu.VMEM((ROWS_A + 1, 1, 128), _i32),
                       pltpu.VMEM((ROWS_A + 1, 1, 128), _i32),
                       pltpu.VMEM((2048,), _f32),
                       pltpu.SemaphoreType.DMA,
                       pltpu.SemaphoreType.DMA],
        compiler_params=pltpu.CompilerParams(needs_layout_passes=False),
    )
    return f(recf, src3d, dst3d)


EBLK = 1280


def _edge_body(red_ref, w1_ref, b1_ref, w2_ref,
               yct_ref, sa_ref, sb_ref, snm_ref, sm_ref, g_ref):
    rdt = lax.transpose(red_ref[...], (1, 0))
    rx = rdt[0:1, :]
    ry = rdt[1:2, :]
    rz = rdt[2:3, :]
    d2 = rx * rx + ry * ry + rz * rz + 1e-12
    dist = jnp.sqrt(d2)
    inv = 1.0 / dist
    fc = 0.5 * (jnp.cos(jnp.pi * jnp.minimum(dist, RC) / RC) + 1.0)
    fc = jnp.where(dist < RC, fc, 0.0)

    h = jnp.tanh(w1_ref[...] * dist + b1_ref[...])
    rn = _mm(w2_ref[...], h) * fc

    x = rx * inv
    y = ry * inv
    z = rz * inv
    xx, yy, zz = x * x, y * y, z * z
    xy, yz, xz = x * y, y * z, x * z
    mono = (jnp.ones_like(x), x, y, z, xy, yz, xz, xx, yy, zz,
            xx * y, yy * y, xy * z, yz * z, zz * z, x * zz,
            xx * z, yy * z, xx * x, x * yy)
    yct = yct_ref[...]
    ysph = yct[:, 0:1] * mono[0]
    for t in range(1, 20):
        ysph = ysph + yct[:, t:t + 1] * mono[t]

    f = jnp.concatenate([rdt, rn, ysph], axis=0)
    fe = lax.transpose(f, (1, 0))
    rnq = _mm(fe, sa_ref[...]) * _mm(fe, sb_ref[...])
    g = _mm(rnq, snm_ref[...]) * _mm(fe, sm_ref[...])
    g_ref[...] = g


def _tc_edges(redge, W1, b1, W2):
    nblk = E_EDGES // EBLK
    consts = [jnp.asarray(a) for a in (_YC.T, _SA, _SB, _SNM, _SM)]
    cspecs = [pl.BlockSpec(a.shape, lambda i: (0, 0)) for a in consts]
    return pl.pallas_call(
        _edge_body,
        grid=(nblk,),
        in_specs=[
            pl.BlockSpec((EBLK, 8), lambda i: (i, 0)),
            pl.BlockSpec((H, 1), lambda i: (0, 0)),
            pl.BlockSpec((H, 1), lambda i: (0, 0)),
            pl.BlockSpec((NPS, H), lambda i: (0, 0)),
        ] + cspecs,
        out_specs=pl.BlockSpec((EBLK, FTOT), lambda i: (i, 0)),
        out_shape=jax.ShapeDtypeStruct((E_EDGES, FTOT), _f32),
    )(redge, W1.reshape(H, 1), b1.reshape(H, 1), W2.T, *consts)


def _scatter_body(g_hbm, src3d, zeros_hbm, coeff_hbm, part_hbm,
                  idxrow, idxbuf, gbuf2, sem0, sem1, acc):
    c = lax.axis_index("c")
    t = lax.axis_index("s")

    def zero_acc():
        pltpu.sync_copy(zeros_hbm, acc.at[pl.ds(t * ZCH, ZCH)])

        @pl.when(t == 0)
        def _():
            pltpu.sync_copy(zeros_hbm.at[pl.ds(0, N_ATOMS - NS * ZCH)],
                            acc.at[pl.ds(NS * ZCH, N_ATOMS - NS * ZCH)])

    def do_row(row, col):
        pltpu.sync_copy(src3d.at[row], idxrow)
        pltpu.sync_copy(g_hbm.at[pl.ds(row * 128, 128), pl.ds(col, 128)],
                        gbuf2.at[0])
        pltpu.sync_copy(gbuf2.at[0], acc.at[idxrow.at[0]], add=True)

    def run_rows(start, n, col):
        pltpu.sync_copy(src3d.at[pl.ds(start, n)], idxbuf.at[pl.ds(0, n)])

        def pair(j, _):
            r0 = start + 2 * j
            c0 = pltpu.async_copy(
                g_hbm.at[pl.ds(r0 * 128, 128), pl.ds(col, 128)],
                gbuf2.at[0], sem0)
            c1 = pltpu.async_copy(
                g_hbm.at[pl.ds((r0 + 1) * 128, 128), pl.ds(col, 128)],
                gbuf2.at[1], sem1)
            c0.wait()
            pltpu.sync_copy(gbuf2.at[0], acc.at[idxbuf.at[2 * j, 0]],
                            add=True)
            c1.wait()
            pltpu.sync_copy(gbuf2.at[1], acc.at[idxbuf.at[2 * j + 1, 0]],
                            add=True)
            return 0

        lax.fori_loop(0, n // 2, pair, 0)
        if n % 2:
            r = start + n - 1
            pltpu.sync_copy(
                g_hbm.at[pl.ds(r * 128, 128), pl.ds(col, 128)], gbuf2.at[0])
            pltpu.sync_copy(gbuf2.at[0], acc.at[idxbuf.at[n - 1, 0]],
                            add=True)

    zero_acc()
    plsc.subcore_barrier()
    run_rows(t * ROWS_C, ROWS_C, c * 128)

    @pl.when(t < EXTRA_C)
    def _():
        do_row(NS * ROWS_C + t, c * 128)

    plsc.subcore_barrier()
    pltpu.sync_copy(acc.at[pl.ds(t * ZCH, ZCH)],
                    coeff_hbm.at[pl.ds(t * ZCH, ZCH), pl.ds(c * 128, 128)])

    @pl.when(t == 0)
    def _():
        pltpu.sync_copy(
            acc.at[pl.ds(NS * ZCH, N_ATOMS - NS * ZCH)],
            coeff_hbm.at[pl.ds(NS * ZCH, N_ATOMS - NS * ZCH),
                         pl.ds(c * 128, 128)])

    plsc.subcore_barrier()
    zero_acc()
    plsc.subcore_barrier()
    hrows = HROWS // NS
    run_rows(c * HROWS + t * hrows, hrows, 2 * 128)

    @pl.when(t < HROWS - NS * hrows)
    def _():
        do_row(c * HROWS + NS * hrows + t, 2 * 128)

    plsc.subcore_barrier()
    pltpu.sync_copy(acc.at[pl.ds(t * ZCH, ZCH)],
                    part_hbm.at[c, pl.ds(t * ZCH, ZCH)])

    @pl.when(t == 0)
    def _():
        pltpu.sync_copy(acc.at[pl.ds(NS * ZCH, N_ATOMS - NS * ZCH)],
                        part_hbm.at[c, pl.ds(NS * ZCH, N_ATOMS - NS * ZCH)])


def _sc_scatter(g, src3d, zeros):
    mesh = plsc.VectorSubcoreMesh(core_axis_name="c", subcore_axis_name="s")
    f = pl.kernel(
        _scatter_body,
        out_type=[jax.ShapeDtypeStruct((N_ATOMS, 256), _f32),
                  jax.ShapeDtypeStruct((NC, N_ATOMS, 128), _f32)],
        mesh=mesh,
        scratch_types=[pltpu.VMEM((1, 128), _i32),
                       pltpu.VMEM((ROWS_C, 1, 128), _i32),
                       pltpu.VMEM((2, 128, 128), _f32),
                       pltpu.SemaphoreType.DMA,
                       pltpu.SemaphoreType.DMA,
                       pltpu.VMEM_SHARED((N_ATOMS, 128), _f32)],
    )
    return f(g, src3d, zeros)


ABLK = 400


def _ps_body(c_ref, pa_ref, pb_ref, p01_ref, p2_ref, r_ref, t_ref, o_ref):
    c2 = pa_ref[0] + pb_ref[0]
    d = _mm(c_ref[...], p01_ref[...]) + _mm(c2, p2_ref[...])
    rsel = r_ref[...]
    tsel = t_ref[...]
    plane = 0
    for l in range(LMAX + 1):
        m = 2 * l + 1
        scale = float(1.0 / np.sqrt(2 * l + 1))
        acc = None
        for mm in range(m):
            a = d[:, (plane + mm) * NPS:(plane + mm + 1) * NPS]
            x1 = _mm(a, rsel)
            x2 = _mm(a, tsel)
            acc = x1 * x2 if acc is None else acc + x1 * x2
        o_ref[:, l * NPS * NPS:(l + 1) * NPS * NPS] = acc * scale
        plane += m


def _tc_ps(coeff256, part):
    nblk = N_ATOMS // ABLK
    p01 = jnp.asarray(_PMAT[:256, :])
    p2 = jnp.asarray(_PMAT[256:, :])
    rsel = jnp.asarray(_RSEL)
    tsel = jnp.asarray(_TSEL)
    return pl.pallas_call(
        _ps_body,
        grid=(nblk,),
        in_specs=[
            pl.BlockSpec((ABLK, 256), lambda i: (i, 0)),
            pl.BlockSpec((1, ABLK, 128), lambda i: (0, i, 0)),
            pl.BlockSpec((1, ABLK, 128), lambda i: (1, i, 0)),
            pl.BlockSpec(p01.shape, lambda i: (0, 0)),
            pl.BlockSpec(p2.shape, lambda i: (0, 0)),
            pl.BlockSpec(rsel.shape, lambda i: (0, 0)),
            pl.BlockSpec(tsel.shape, lambda i: (0, 0)),
        ],
        out_specs=pl.BlockSpec((ABLK, OUT_W), lambda i: (i, 0)),
        out_shape=jax.ShapeDtypeStruct((N_ATOMS, OUT_W), _f32),
    )(coeff256, part, part, p01, p2, rsel, tsel)


def kernel(positions, species, pairs, W1, b1, W2, alch):
    wqt = jnp.take(alch, species, axis=0)
    zero_col = jnp.zeros((N_ATOMS, 1), _f32)
    recf = jnp.concatenate([wqt, zero_col, positions], axis=1).reshape(-1)
    src3d = pairs[0].reshape(ROWS, 1, 128)
    dst3d = pairs[1].reshape(ROWS, 1, 128)

    redge = _sc_gather(recf, src3d, dst3d).reshape(E_EDGES, 8)
    g = _tc_edges(redge, W1, b1, W2)
    zeros = jnp.zeros((ZCH, 128), _f32)
    coeff256, part = _sc_scatter(g, src3d, zeros)
    return _tc_ps(coeff256, part)


# speedup vs baseline: 131.2442x; 1.0439x over previous
import numpy as np

import jax
import jax.numpy as jnp
from jax import lax
from jax.experimental import pallas as pl
from jax.experimental.pallas import tpu as pltpu
from jax.experimental.pallas import tpu_sc as plsc

N_ATOMS = 10000
E_EDGES = 160000
Q = 4
NMAX = 6
LMAX = 3
RC = 5.0
H = 32
MOFF = (0, 1, 4, 9)
F96 = 96
FTOT = Q * F96
NPS = 24
OUT_W = (LMAX + 1) * NPS * NPS

NC, NS = 2, 16
NW = NC * NS
ROWS = E_EDGES // 128
ROWS_A = ROWS // NW
EXTRA_A = ROWS - ROWS_A * NW
ROWS_C = ROWS // NS
EXTRA_C = ROWS - ROWS_C * NS
ZCH = 624
HROWS = ROWS // NC

_f32 = jnp.float32
_i32 = jnp.int32


def _np_consts():
    c0 = 0.28209479177387814
    c1 = 0.4886025119029199
    c2a, c2b, c2c = 1.0925484305920792, 0.31539156525252005, 0.5462742152960396
    c3a, c3b, c3c, c3d = (0.5900435899266435, 2.890611442640554,
                          0.4570457994644658, 0.3731763325901154)
    yc = np.zeros((20, 16), np.float32)
    yc[0, 0] = c0
    yc[2, 1] = c1
    yc[3, 2] = c1
    yc[1, 3] = c1
    yc[4, 4] = c2a
    yc[5, 5] = c2a
    yc[9, 6] = 3.0 * c2b
    yc[0, 6] = -c2b
    yc[6, 7] = c2a
    yc[7, 8] = c2c
    yc[8, 8] = -c2c
    yc[10, 9] = 3.0 * c3a
    yc[11, 9] = -c3a
    yc[12, 10] = c3b
    yc[13, 11] = 5.0 * c3c
    yc[2, 11] = -c3c
    yc[14, 12] = 5.0 * c3d
    yc[3, 12] = -3.0 * c3d
    yc[15, 13] = 5.0 * c3c
    yc[1, 13] = -c3c
    yc[16, 14] = 1.445305721320277
    yc[17, 14] = -1.445305721320277
    yc[18, 15] = c3a
    yc[19, 15] = -3.0 * c3a

    sa = np.zeros((48, F96), np.float32)
    sb = np.zeros((48, F96), np.float32)
    for q in range(Q):
        for ln in range(NPS):
            sa[4 + q, q * NPS + ln] = 1.0
            sb[8 + ln, q * NPS + ln] = 1.0
    snm = np.zeros((F96, FTOT), np.float32)
    sm = np.zeros((48, FTOT), np.float32)
    for q in range(Q):
        for l in range(LMAX + 1):
            m = 2 * l + 1
            for n in range(NMAX):
                for mm in range(m):
                    col = q * F96 + 6 * MOFF[l] + n * m + mm
                    snm[q * NPS + l * NMAX + n, col] = 1.0
                    sm[32 + MOFF[l] + mm, col] = 1.0
    pmat = np.zeros((FTOT, FTOT), np.float32)
    plane = 0
    for l in range(LMAX + 1):
        m = 2 * l + 1
        for mm in range(m):
            for q in range(Q):
                for n in range(NMAX):
                    col = q * F96 + 6 * MOFF[l] + n * m + mm
                    pmat[col, plane * NPS + q * NMAX + n] = 1.0
            plane += 1
    rsel = np.zeros((NPS, NPS * NPS), np.float32)
    tsel = np.zeros((NPS, NPS * NPS), np.float32)
    for a in range(NPS):
        for b in range(NPS):
            rsel[a, a * NPS + b] = 1.0
            tsel[b, a * NPS + b] = 1.0
    return yc, sa, sb, snm, sm, pmat, rsel, tsel


_YC, _SA, _SB, _SNM, _SM, _PMAT, _RSEL, _TSEL = _np_consts()


def _mm(a, b):
    return lax.dot_general(a, b, (((1,), (0,)), ((), ())),
                           preferred_element_type=_f32)


def _gather_body(recf, src3d, dst3d, redge,
                 table, sidxb, didxb, drow2, wsem0, wsem1):
    wid = lax.axis_index("s") * NC + lax.axis_index("c")
    pltpu.sync_copy(recf, table)
    lanes = lax.iota(_i32, 16)
    start = wid * ROWS_A
    pltpu.sync_copy(src3d.at[pl.ds(start, ROWS_A)],
                    sidxb.at[pl.ds(0, ROWS_A)])
    pltpu.sync_copy(dst3d.at[pl.ds(start, ROWS_A)],
                    didxb.at[pl.ds(0, ROWS_A)])

    @pl.when(wid < EXTRA_A)
    def _():
        pltpu.sync_copy(src3d.at[pl.ds(NW * ROWS_A + wid, 1)],
                        sidxb.at[pl.ds(ROWS_A, 1)])
        pltpu.sync_copy(dst3d.at[pl.ds(NW * ROWS_A + wid, 1)],
                        didxb.at[pl.ds(ROWS_A, 1)])

    def build(j, slot):
        for k in range(8):
            s16 = sidxb[j, 0, pl.ds(k * 16, 16)] * 8
            d16 = didxb[j, 0, pl.ds(k * 16, 16)] * 8
            dl8 = slot * 1024 + (lanes + k * 16) * 8
            for comp in range(3):
                vs = plsc.load_gather(table, [s16 + (5 + comp)])
                vd = plsc.load_gather(table, [d16 + (5 + comp)])
                plsc.store_scatter(drow2, [dl8 + comp], vd - vs)
            for comp in range(4):
                vd = plsc.load_gather(table, [d16 + comp])
                plsc.store_scatter(drow2, [dl8 + 4 + comp], vd)

    def pair(i, _):
        r0 = start + 2 * i
        build(2 * i, 0)
        c0 = pltpu.async_copy(drow2.at[pl.ds(0, 1024)],
                              redge.at[pl.ds(r0 * 1024, 1024)], wsem0)
        build(2 * i + 1, 1)
        c1 = pltpu.async_copy(drow2.at[pl.ds(1024, 1024)],
                              redge.at[pl.ds((r0 + 1) * 1024, 1024)], wsem1)
        c0.wait()
        c1.wait()
        return 0

    lax.fori_loop(0, ROWS_A // 2, pair, 0)
    if ROWS_A % 2:
        build(ROWS_A - 1, 0)
        pltpu.sync_copy(drow2.at[pl.ds(0, 1024)],
                        redge.at[pl.ds((start + ROWS_A - 1) * 1024, 1024)])

    @pl.when(wid < EXTRA_A)
    def _():
        build(ROWS_A, 0)
        pltpu.sync_copy(
            drow2.at[pl.ds(0, 1024)],
            redge.at[pl.ds((NW * ROWS_A + wid) * 1024, 1024)])


def _sc_gather(recf, src3d, dst3d):
    mesh = plsc.VectorSubcoreMesh(core_axis_name="c", subcore_axis_name="s")
    f = pl.kernel(
        _gather_body,
        out_type=jax.ShapeDtypeStruct((E_EDGES * 8,), _f32),
        mesh=mesh,
        scratch_types=[pltpu.VMEM((N_ATOMS * 8,), _f32),
                       pltpu.VMEM((ROWS_A + 1, 1, 128), _i32),
                       pltpu.VMEM((ROWS_A + 1, 1, 128), _i32),
                       pltpu.VMEM((2048,), _f32),
                       pltpu.SemaphoreType.DMA,
                       pltpu.SemaphoreType.DMA],
        compiler_params=pltpu.CompilerParams(needs_layout_passes=False),
    )
    return f(recf, src3d, dst3d)


EBLK = 3200


def _edge_body(red_ref, w1_ref, b1_ref, w2_ref,
               yct_ref, sa_ref, sb_ref, snm_ref, sm_ref, g_ref):
    rdt = lax.transpose(red_ref[...], (1, 0))
    rx = rdt[0:1, :]
    ry = rdt[1:2, :]
    rz = rdt[2:3, :]
    d2 = rx * rx + ry * ry + rz * rz + 1e-12
    dist = jnp.sqrt(d2)
    inv = 1.0 / dist
    fc = 0.5 * (jnp.cos(jnp.pi * jnp.minimum(dist, RC) / RC) + 1.0)
    fc = jnp.where(dist < RC, fc, 0.0)

    h = jnp.tanh(w1_ref[...] * dist + b1_ref[...])
    rn = _mm(w2_ref[...], h) * fc

    x = rx * inv
    y = ry * inv
    z = rz * inv
    xx, yy, zz = x * x, y * y, z * z
    xy, yz, xz = x * y, y * z, x * z
    mono = (jnp.ones_like(x), x, y, z, xy, yz, xz, xx, yy, zz,
            xx * y, yy * y, xy * z, yz * z, zz * z, x * zz,
            xx * z, yy * z, xx * x, x * yy)
    yct = yct_ref[...]
    ysph = yct[:, 0:1] * mono[0]
    for t in range(1, 20):
        ysph = ysph + yct[:, t:t + 1] * mono[t]

    f = jnp.concatenate([rdt, rn, ysph], axis=0)
    fe = lax.transpose(f, (1, 0))
    rnq = _mm(fe, sa_ref[...]) * _mm(fe, sb_ref[...])
    g = _mm(rnq, snm_ref[...]) * _mm(fe, sm_ref[...])
    g_ref[...] = g


def _tc_edges(redge, W1, b1, W2):
    nblk = E_EDGES // EBLK
    consts = [jnp.asarray(a) for a in (_YC.T, _SA, _SB, _SNM, _SM)]
    cspecs = [pl.BlockSpec(a.shape, lambda i: (0, 0)) for a in consts]
    return pl.pallas_call(
        _edge_body,
        grid=(nblk,),
        in_specs=[
            pl.BlockSpec((EBLK, 8), lambda i: (i, 0)),
            pl.BlockSpec((H, 1), lambda i: (0, 0)),
            pl.BlockSpec((H, 1), lambda i: (0, 0)),
            pl.BlockSpec((NPS, H), lambda i: (0, 0)),
        ] + cspecs,
        out_specs=pl.BlockSpec((EBLK, FTOT), lambda i: (i, 0)),
        out_shape=jax.ShapeDtypeStruct((E_EDGES, FTOT), _f32),
    )(redge, W1.reshape(H, 1), b1.reshape(H, 1), W2.T, *consts)


def _scatter_body(g_hbm, src3d, zeros_hbm, coeff_hbm, part_hbm,
                  idxrow, idxbuf, gbuf2, sem0, sem1, acc):
    c = lax.axis_index("c")
    t = lax.axis_index("s")

    def zero_acc():
        pltpu.sync_copy(zeros_hbm, acc.at[pl.ds(t * ZCH, ZCH)])

        @pl.when(t == 0)
        def _():
            pltpu.sync_copy(zeros_hbm.at[pl.ds(0, N_ATOMS - NS * ZCH)],
                            acc.at[pl.ds(NS * ZCH, N_ATOMS - NS * ZCH)])

    def do_row(row, col):
        pltpu.sync_copy(src3d.at[row], idxrow)
        pltpu.sync_copy(g_hbm.at[pl.ds(row * 128, 128), pl.ds(col, 128)],
                        gbuf2.at[0])
        pltpu.sync_copy(gbuf2.at[0], acc.at[idxrow.at[0]], add=True)

    def run_rows(start, n, col):
        pltpu.sync_copy(src3d.at[pl.ds(start, n)], idxbuf.at[pl.ds(0, n)])

        def pair(j, _):
            r0 = start + 2 * j
            c0 = pltpu.async_copy(
                g_hbm.at[pl.ds(r0 * 128, 128), pl.ds(col, 128)],
                gbuf2.at[0], sem0)
            c1 = pltpu.async_copy(
                g_hbm.at[pl.ds((r0 + 1) * 128, 128), pl.ds(col, 128)],
                gbuf2.at[1], sem1)
            c0.wait()
            pltpu.sync_copy(gbuf2.at[0], acc.at[idxbuf.at[2 * j, 0]],
                            add=True)
            c1.wait()
            pltpu.sync_copy(gbuf2.at[1], acc.at[idxbuf.at[2 * j + 1, 0]],
                            add=True)
            return 0

        lax.fori_loop(0, n // 2, pair, 0)
        if n % 2:
            r = start + n - 1
            pltpu.sync_copy(
                g_hbm.at[pl.ds(r * 128, 128), pl.ds(col, 128)], gbuf2.at[0])
            pltpu.sync_copy(gbuf2.at[0], acc.at[idxbuf.at[n - 1, 0]],
                            add=True)

    zero_acc()
    plsc.subcore_barrier()
    run_rows(t * ROWS_C, ROWS_C, c * 128)

    @pl.when(t < EXTRA_C)
    def _():
        do_row(NS * ROWS_C + t, c * 128)

    plsc.subcore_barrier()
    pltpu.sync_copy(acc.at[pl.ds(t * ZCH, ZCH)],
                    coeff_hbm.at[pl.ds(t * ZCH, ZCH), pl.ds(c * 128, 128)])

    @pl.when(t == 0)
    def _():
        pltpu.sync_copy(
            acc.at[pl.ds(NS * ZCH, N_ATOMS - NS * ZCH)],
            coeff_hbm.at[pl.ds(NS * ZCH, N_ATOMS - NS * ZCH),
                         pl.ds(c * 128, 128)])

    plsc.subcore_barrier()
    zero_acc()
    plsc.subcore_barrier()
    hrows = HROWS // NS
    run_rows(c * HROWS + t * hrows, hrows, 2 * 128)

    @pl.when(t < HROWS - NS * hrows)
    def _():
        do_row(c * HROWS + NS * hrows + t, 2 * 128)

    plsc.subcore_barrier()
    pltpu.sync_copy(acc.at[pl.ds(t * ZCH, ZCH)],
                    part_hbm.at[c, pl.ds(t * ZCH, ZCH)])

    @pl.when(t == 0)
    def _():
        pltpu.sync_copy(acc.at[pl.ds(NS * ZCH, N_ATOMS - NS * ZCH)],
                        part_hbm.at[c, pl.ds(NS * ZCH, N_ATOMS - NS * ZCH)])


def _sc_scatter(g, src3d, zeros):
    mesh = plsc.VectorSubcoreMesh(core_axis_name="c", subcore_axis_name="s")
    f = pl.kernel(
        _scatter_body,
        out_type=[jax.ShapeDtypeStruct((N_ATOMS, 256), _f32),
                  jax.ShapeDtypeStruct((NC, N_ATOMS, 128), _f32)],
        mesh=mesh,
        scratch_types=[pltpu.VMEM((1, 128), _i32),
                       pltpu.VMEM((ROWS_C, 1, 128), _i32),
                       pltpu.VMEM((2, 128, 128), _f32),
                       pltpu.SemaphoreType.DMA,
                       pltpu.SemaphoreType.DMA,
                       pltpu.VMEM_SHARED((N_ATOMS, 128), _f32)],
    )
    return f(g, src3d, zeros)


ABLK = 1000


def _ps_body(c_ref, pa_ref, pb_ref, p01_ref, p2_ref, r_ref, t_ref, o_ref):
    c2 = pa_ref[0] + pb_ref[0]
    d = _mm(c_ref[...], p01_ref[...]) + _mm(c2, p2_ref[...])
    rsel = r_ref[...]
    tsel = t_ref[...]
    plane = 0
    for l in range(LMAX + 1):
        m = 2 * l + 1
        scale = float(1.0 / np.sqrt(2 * l + 1))
        acc = None
        for mm in range(m):
            a = d[:, (plane + mm) * NPS:(plane + mm + 1) * NPS]
            x1 = _mm(a, rsel)
            x2 = _mm(a, tsel)
            acc = x1 * x2 if acc is None else acc + x1 * x2
        o_ref[:, l * NPS * NPS:(l + 1) * NPS * NPS] = acc * scale
        plane += m


def _tc_ps(coeff256, part):
    nblk = N_ATOMS // ABLK
    p01 = jnp.asarray(_PMAT[:256, :])
    p2 = jnp.asarray(_PMAT[256:, :])
    rsel = jnp.asarray(_RSEL)
    tsel = jnp.asarray(_TSEL)
    return pl.pallas_call(
        _ps_body,
        grid=(nblk,),
        in_specs=[
            pl.BlockSpec((ABLK, 256), lambda i: (i, 0)),
            pl.BlockSpec((1, ABLK, 128), lambda i: (0, i, 0)),
            pl.BlockSpec((1, ABLK, 128), lambda i: (1, i, 0)),
            pl.BlockSpec(p01.shape, lambda i: (0, 0)),
            pl.BlockSpec(p2.shape, lambda i: (0, 0)),
            pl.BlockSpec(rsel.shape, lambda i: (0, 0)),
            pl.BlockSpec(tsel.shape, lambda i: (0, 0)),
        ],
        out_specs=pl.BlockSpec((ABLK, OUT_W), lambda i: (i, 0)),
        out_shape=jax.ShapeDtypeStruct((N_ATOMS, OUT_W), _f32),
    )(coeff256, part, part, p01, p2, rsel, tsel)


def kernel(positions, species, pairs, W1, b1, W2, alch):
    wqt = jnp.take(alch, species, axis=0)
    zero_col = jnp.zeros((N_ATOMS, 1), _f32)
    recf = jnp.concatenate([wqt, zero_col, positions], axis=1).reshape(-1)
    src3d = pairs[0].reshape(ROWS, 1, 128)
    dst3d = pairs[1].reshape(ROWS, 1, 128)

    redge = _sc_gather(recf, src3d, dst3d).reshape(E_EDGES, 8)
    g = _tc_edges(redge, W1, b1, W2)
    zeros = jnp.zeros((ZCH, 128), _f32)
    coeff256, part = _sc_scatter(g, src3d, zeros)
    return _tc_ps(coeff256, part)


# speedup vs baseline: 131.3375x; 1.0007x over previous
import numpy as np

import jax
import jax.numpy as jnp
from jax import lax
from jax.experimental import pallas as pl
from jax.experimental.pallas import tpu as pltpu
from jax.experimental.pallas import tpu_sc as plsc

N_ATOMS = 10000
E_EDGES = 160000
Q = 4
NMAX = 6
LMAX = 3
RC = 5.0
H = 32
MOFF = (0, 1, 4, 9)
F96 = 96
FTOT = Q * F96
NPS = 24
OUT_W = (LMAX + 1) * NPS * NPS

NC, NS = 2, 16
NW = NC * NS
ROWS = E_EDGES // 128
ROWS_A = ROWS // NW
EXTRA_A = ROWS - ROWS_A * NW
ROWS_C = ROWS // NS
EXTRA_C = ROWS - ROWS_C * NS
ZCH = 624
HROWS = ROWS // NC

_f32 = jnp.float32
_i32 = jnp.int32


def _np_consts():
    c0 = 0.28209479177387814
    c1 = 0.4886025119029199
    c2a, c2b, c2c = 1.0925484305920792, 0.31539156525252005, 0.5462742152960396
    c3a, c3b, c3c, c3d = (0.5900435899266435, 2.890611442640554,
                          0.4570457994644658, 0.3731763325901154)
    yc = np.zeros((20, 16), np.float32)
    yc[0, 0] = c0
    yc[2, 1] = c1
    yc[3, 2] = c1
    yc[1, 3] = c1
    yc[4, 4] = c2a
    yc[5, 5] = c2a
    yc[9, 6] = 3.0 * c2b
    yc[0, 6] = -c2b
    yc[6, 7] = c2a
    yc[7, 8] = c2c
    yc[8, 8] = -c2c
    yc[10, 9] = 3.0 * c3a
    yc[11, 9] = -c3a
    yc[12, 10] = c3b
    yc[13, 11] = 5.0 * c3c
    yc[2, 11] = -c3c
    yc[14, 12] = 5.0 * c3d
    yc[3, 12] = -3.0 * c3d
    yc[15, 13] = 5.0 * c3c
    yc[1, 13] = -c3c
    yc[16, 14] = 1.445305721320277
    yc[17, 14] = -1.445305721320277
    yc[18, 15] = c3a
    yc[19, 15] = -3.0 * c3a

    sa = np.zeros((48, F96), np.float32)
    sb = np.zeros((48, F96), np.float32)
    for q in range(Q):
        for ln in range(NPS):
            sa[4 + q, q * NPS + ln] = 1.0
            sb[8 + ln, q * NPS + ln] = 1.0
    snm = np.zeros((F96, FTOT), np.float32)
    sm = np.zeros((48, FTOT), np.float32)
    for q in range(Q):
        for l in range(LMAX + 1):
            m = 2 * l + 1
            for n in range(NMAX):
                for mm in range(m):
                    col = q * F96 + 6 * MOFF[l] + n * m + mm
                    snm[q * NPS + l * NMAX + n, col] = 1.0
                    sm[32 + MOFF[l] + mm, col] = 1.0
    pmat = np.zeros((FTOT, FTOT), np.float32)
    plane = 0
    for l in range(LMAX + 1):
        m = 2 * l + 1
        for mm in range(m):
            for q in range(Q):
                for n in range(NMAX):
                    col = q * F96 + 6 * MOFF[l] + n * m + mm
                    pmat[col, plane * NPS + q * NMAX + n] = 1.0
            plane += 1
    rsel = np.zeros((NPS, NPS * NPS), np.float32)
    tsel = np.zeros((NPS, NPS * NPS), np.float32)
    for a in range(NPS):
        for b in range(NPS):
            rsel[a, a * NPS + b] = 1.0
            tsel[b, a * NPS + b] = 1.0
    return yc, sa, sb, snm, sm, pmat, rsel, tsel


_YC, _SA, _SB, _SNM, _SM, _PMAT, _RSEL, _TSEL = _np_consts()


def _mm(a, b):
    return lax.dot_general(a, b, (((1,), (0,)), ((), ())),
                           preferred_element_type=_f32)


def _gather_body(recf, src3d, dst3d, redge,
                 table, sidxb, didxb, drow2, wsem0, wsem1):
    wid = lax.axis_index("s") * NC + lax.axis_index("c")
    pltpu.sync_copy(recf, table)
    lanes = lax.iota(_i32, 16)
    start = wid * ROWS_A
    pltpu.sync_copy(src3d.at[pl.ds(start, ROWS_A)],
                    sidxb.at[pl.ds(0, ROWS_A)])
    pltpu.sync_copy(dst3d.at[pl.ds(start, ROWS_A)],
                    didxb.at[pl.ds(0, ROWS_A)])

    @pl.when(wid < EXTRA_A)
    def _():
        pltpu.sync_copy(src3d.at[pl.ds(NW * ROWS_A + wid, 1)],
                        sidxb.at[pl.ds(ROWS_A, 1)])
        pltpu.sync_copy(dst3d.at[pl.ds(NW * ROWS_A + wid, 1)],
                        didxb.at[pl.ds(ROWS_A, 1)])

    def build(j, slot):
        for k in range(8):
            s16 = sidxb[j, 0, pl.ds(k * 16, 16)] * 8
            d16 = didxb[j, 0, pl.ds(k * 16, 16)] * 8
            dl8 = slot * 1024 + (lanes + k * 16) * 8
            for comp in range(3):
                vs = plsc.load_gather(table, [s16 + (5 + comp)])
                vd = plsc.load_gather(table, [d16 + (5 + comp)])
                plsc.store_scatter(drow2, [dl8 + comp], vd - vs)
            for comp in range(4):
                vd = plsc.load_gather(table, [d16 + comp])
                plsc.store_scatter(drow2, [dl8 + 4 + comp], vd)

    def pair(i, _):
        r0 = start + 2 * i
        build(2 * i, 0)
        c0 = pltpu.async_copy(drow2.at[pl.ds(0, 1024)],
                              redge.at[pl.ds(r0 * 1024, 1024)], wsem0)
        build(2 * i + 1, 1)
        c1 = pltpu.async_copy(drow2.at[pl.ds(1024, 1024)],
                              redge.at[pl.ds((r0 + 1) * 1024, 1024)], wsem1)
        c0.wait()
        c1.wait()
        return 0

    lax.fori_loop(0, ROWS_A // 2, pair, 0)
    if ROWS_A % 2:
        build(ROWS_A - 1, 0)
        pltpu.sync_copy(drow2.at[pl.ds(0, 1024)],
                        redge.at[pl.ds((start + ROWS_A - 1) * 1024, 1024)])

    @pl.when(wid < EXTRA_A)
    def _():
        build(ROWS_A, 0)
        pltpu.sync_copy(
            drow2.at[pl.ds(0, 1024)],
            redge.at[pl.ds((NW * ROWS_A + wid) * 1024, 1024)])


def _sc_gather(recf, src3d, dst3d):
    mesh = plsc.VectorSubcoreMesh(core_axis_name="c", subcore_axis_name="s")
    f = pl.kernel(
        _gather_body,
        out_type=jax.ShapeDtypeStruct((E_EDGES * 8,), _f32),
        mesh=mesh,
        scratch_types=[pltpu.VMEM((N_ATOMS * 8,), _f32),
                       pltpu.VMEM((ROWS_A + 1, 1, 128), _i32),
                       pltpu.VMEM((ROWS_A + 1, 1, 128), _i32),
                       pltpu.VMEM((2048,), _f32),
                       pltpu.SemaphoreType.DMA,
                       pltpu.SemaphoreType.DMA],
        compiler_params=pltpu.CompilerParams(needs_layout_passes=False),
    )
    return f(recf, src3d, dst3d)


EBLK = 3200


def _edge_body(red_ref, w1_ref, b1_ref, w2_ref,
               yct_ref, sa_ref, sb_ref, snm_ref, sm_ref, g_ref):
    rdt = lax.transpose(red_ref[...], (1, 0))
    rx = rdt[0:1, :]
    ry = rdt[1:2, :]
    rz = rdt[2:3, :]
    d2 = rx * rx + ry * ry + rz * rz + 1e-12
    dist = jnp.sqrt(d2)
    inv = 1.0 / dist
    fc = 0.5 * (jnp.cos(jnp.pi * jnp.minimum(dist, RC) / RC) + 1.0)
    fc = jnp.where(dist < RC, fc, 0.0)

    h = jnp.tanh(w1_ref[...] * dist + b1_ref[...])
    rn = _mm(w2_ref[...], h) * fc

    x = rx * inv
    y = ry * inv
    z = rz * inv
    xx, yy, zz = x * x, y * y, z * z
    xy, yz, xz = x * y, y * z, x * z
    mono = (jnp.ones_like(x), x, y, z, xy, yz, xz, xx, yy, zz,
            xx * y, yy * y, xy * z, yz * z, zz * z, x * zz,
            xx * z, yy * z, xx * x, x * yy)
    yct = yct_ref[...]
    ysph = yct[:, 0:1] * mono[0]
    for t in range(1, 20):
        ysph = ysph + yct[:, t:t + 1] * mono[t]

    f = jnp.concatenate([rdt, rn, ysph], axis=0)
    fe = lax.transpose(f, (1, 0))
    rnq = _mm(fe, sa_ref[...]) * _mm(fe, sb_ref[...])
    g = _mm(rnq, snm_ref[...]) * _mm(fe, sm_ref[...])
    g_ref[...] = g


def _tc_edges(redge, W1, b1, W2):
    nblk = E_EDGES // EBLK
    consts = [jnp.asarray(a) for a in (_YC.T, _SA, _SB, _SNM, _SM)]
    cspecs = [pl.BlockSpec(a.shape, lambda i: (0, 0)) for a in consts]
    return pl.pallas_call(
        _edge_body,
        grid=(nblk,),
        in_specs=[
            pl.BlockSpec((EBLK, 8), lambda i: (i, 0)),
            pl.BlockSpec((H, 1), lambda i: (0, 0)),
            pl.BlockSpec((H, 1), lambda i: (0, 0)),
            pl.BlockSpec((NPS, H), lambda i: (0, 0)),
        ] + cspecs,
        out_specs=pl.BlockSpec((EBLK, FTOT), lambda i: (i, 0)),
        out_shape=jax.ShapeDtypeStruct((E_EDGES, FTOT), _f32),
    )(redge, W1.reshape(H, 1), b1.reshape(H, 1), W2.T, *consts)


def _scatter_body(g_hbm, src3d, zeros_hbm, coeff_hbm, part_hbm,
                  idxrow, idxbuf, gbuf2, sem0, sem1, acc):
    c = lax.axis_index("c")
    t = lax.axis_index("s")
    sems = (sem0, sem1)

    def zero_acc():
        pltpu.sync_copy(zeros_hbm, acc.at[pl.ds(t * ZCH, ZCH)])

        @pl.when(t == 0)
        def _():
            pltpu.sync_copy(zeros_hbm.at[pl.ds(0, N_ATOMS - NS * ZCH)],
                            acc.at[pl.ds(NS * ZCH, N_ATOMS - NS * ZCH)])

    def do_row(row, col):
        pltpu.sync_copy(src3d.at[row], idxrow)
        pltpu.sync_copy(g_hbm.at[pl.ds(row * 128, 128), pl.ds(col, 128)],
                        gbuf2.at[0])
        pltpu.sync_copy(gbuf2.at[0], acc.at[idxrow.at[0]], add=True)

    def run_rows(start, n, col):
        pltpu.sync_copy(src3d.at[pl.ds(start, n)], idxbuf.at[pl.ds(0, n)])

        def pair(j, _):
            r0 = start + 2 * j
            c0 = pltpu.async_copy(
                g_hbm.at[pl.ds(r0 * 128, 128), pl.ds(col, 128)],
                gbuf2.at[0], sems[0])
            c1 = pltpu.async_copy(
                g_hbm.at[pl.ds((r0 + 1) * 128, 128), pl.ds(col, 128)],
                gbuf2.at[1], sems[1])
            c0.wait()
            pltpu.sync_copy(gbuf2.at[0], acc.at[idxbuf.at[2 * j, 0]],
                            add=True)
            c1.wait()
            pltpu.sync_copy(gbuf2.at[1], acc.at[idxbuf.at[2 * j + 1, 0]],
                            add=True)
            return 0

        lax.fori_loop(0, n // 2, pair, 0)
        if n % 2:
            r = start + n - 1
            pltpu.sync_copy(
                g_hbm.at[pl.ds(r * 128, 128), pl.ds(col, 128)], gbuf2.at[0])
            pltpu.sync_copy(gbuf2.at[0], acc.at[idxbuf.at[n - 1, 0]],
                            add=True)

    zero_acc()
    plsc.subcore_barrier()
    run_rows(t * ROWS_C, ROWS_C, c * 128)

    @pl.when(t < EXTRA_C)
    def _():
        do_row(NS * ROWS_C + t, c * 128)

    plsc.subcore_barrier()
    pltpu.sync_copy(acc.at[pl.ds(t * ZCH, ZCH)],
                    coeff_hbm.at[pl.ds(t * ZCH, ZCH), pl.ds(c * 128, 128)])

    @pl.when(t == 0)
    def _():
        pltpu.sync_copy(
            acc.at[pl.ds(NS * ZCH, N_ATOMS - NS * ZCH)],
            coeff_hbm.at[pl.ds(NS * ZCH, N_ATOMS - NS * ZCH),
                         pl.ds(c * 128, 128)])

    plsc.subcore_barrier()
    zero_acc()
    plsc.subcore_barrier()
    hrows = HROWS // NS
    run_rows(c * HROWS + t * hrows, hrows, 2 * 128)

    @pl.when(t < HROWS - NS * hrows)
    def _():
        do_row(c * HROWS + NS * hrows + t, 2 * 128)

    plsc.subcore_barrier()
    pltpu.sync_copy(acc.at[pl.ds(t * ZCH, ZCH)],
                    part_hbm.at[c, pl.ds(t * ZCH, ZCH)])

    @pl.when(t == 0)
    def _():
        pltpu.sync_copy(acc.at[pl.ds(NS * ZCH, N_ATOMS - NS * ZCH)],
                        part_hbm.at[c, pl.ds(NS * ZCH, N_ATOMS - NS * ZCH)])


def _sc_scatter(g, src3d, zeros):
    mesh = plsc.VectorSubcoreMesh(core_axis_name="c", subcore_axis_name="s")
    f = pl.kernel(
        _scatter_body,
        out_type=[jax.ShapeDtypeStruct((N_ATOMS, 256), _f32),
                  jax.ShapeDtypeStruct((NC, N_ATOMS, 128), _f32)],
        mesh=mesh,
        scratch_types=[pltpu.VMEM((1, 128), _i32),
                       pltpu.VMEM((ROWS_C, 1, 128), _i32),
                       pltpu.VMEM((2, 128, 128), _f32),
                       pltpu.SemaphoreType.DMA,
                       pltpu.SemaphoreType.DMA,
                       pltpu.VMEM_SHARED((N_ATOMS, 128), _f32)],
    )
    return f(g, src3d, zeros)


ABLK = 1000


def _ps_body(c_ref, pa_ref, pb_ref, p01_ref, p2_ref, r_ref, t_ref, o_ref):
    c2 = pa_ref[0] + pb_ref[0]
    d = _mm(c_ref[...], p01_ref[...]) + _mm(c2, p2_ref[...])
    rsel = r_ref[...]
    tsel = t_ref[...]
    plane = 0
    for l in range(LMAX + 1):
        m = 2 * l + 1
        scale = float(1.0 / np.sqrt(2 * l + 1))
        acc = None
        for mm in range(m):
            a = d[:, (plane + mm) * NPS:(plane + mm + 1) * NPS]
            x1 = _mm(a, rsel)
            x2 = _mm(a, tsel)
            acc = x1 * x2 if acc is None else acc + x1 * x2
        o_ref[:, l * NPS * NPS:(l + 1) * NPS * NPS] = acc * scale
        plane += m


def _tc_ps(coeff256, part):
    nblk = N_ATOMS // ABLK
    p01 = jnp.asarray(_PMAT[:256, :])
    p2 = jnp.asarray(_PMAT[256:, :])
    rsel = jnp.asarray(_RSEL)
    tsel = jnp.asarray(_TSEL)
    return pl.pallas_call(
        _ps_body,
        grid=(nblk,),
        in_specs=[
            pl.BlockSpec((ABLK, 256), lambda i: (i, 0)),
            pl.BlockSpec((1, ABLK, 128), lambda i: (0, i, 0)),
            pl.BlockSpec((1, ABLK, 128), lambda i: (1, i, 0)),
            pl.BlockSpec(p01.shape, lambda i: (0, 0)),
            pl.BlockSpec(p2.shape, lambda i: (0, 0)),
            pl.BlockSpec(rsel.shape, lambda i: (0, 0)),
            pl.BlockSpec(tsel.shape, lambda i: (0, 0)),
        ],
        out_specs=pl.BlockSpec((ABLK, OUT_W), lambda i: (i, 0)),
        out_shape=jax.ShapeDtypeStruct((N_ATOMS, OUT_W), _f32),
    )(coeff256, part, part, p01, p2, rsel, tsel)


def kernel(positions, species, pairs, W1, b1, W2, alch):
    wqt = jnp.take(alch, species, axis=0)
    zero_col = jnp.zeros((N_ATOMS, 1), _f32)
    recf = jnp.concatenate([wqt, zero_col, positions], axis=1).reshape(-1)
    src3d = pairs[0].reshape(ROWS, 1, 128)
    dst3d = pairs[1].reshape(ROWS, 1, 128)

    redge = _sc_gather(recf, src3d, dst3d).reshape(E_EDGES, 8)
    g = _tc_edges(redge, W1, b1, W2)
    zeros = jnp.zeros((ZCH, 128), _f32)
    coeff256, part = _sc_scatter(g, src3d, zeros)
    return _tc_ps(coeff256, part)
